# baseline (speedup 1.0000x reference)
import functools

import jax
import jax.numpy as jnp
from jax import lax
from jax.experimental import pallas as pl
from jax.experimental.pallas import tpu as pltpu
from jax.experimental.pallas import tpu_sc as plsc

N_ATOMS = 50000
N_BONDS = 800000
ATOM_FDIM = 133
BOND_FDIM = 13
HIDDEN = 64
TASKS = 12
DEPTH = 3
N_MOLS = 500

NC = 2
NS = 16
NQ = 4
QW = HIDDEN // NQ
BPS = N_BONDS // NS
CHUNK = 80
NCHUNK = BPS // CHUNK
DROWS = 3128
DLAST = N_ATOMS - (NS - 1) * DROWS
NMOLP = 512

_mesh = plsc.VectorSubcoreMesh(core_axis_name="c", subcore_axis_name="s")
_f32 = jnp.float32
_i32 = jnp.int32


def _zero_acc(zb, acc, s):
    @pl.loop(0, DROWS)
    def _(i):
        zb[i, :] = jnp.zeros((QW,), _f32)

    @pl.when(s < NS - 1)
    def _():
        pltpu.sync_copy(zb, acc.at[pl.ds(s * DROWS, DROWS)])

    @pl.when(s == NS - 1)
    def _():
        pltpu.sync_copy(zb.at[pl.ds(0, DLAST)], acc.at[pl.ds(s * DROWS, DLAST)])


def _dump_acc(acc, s_ref, q, s):
    @pl.when(s < NS - 1)
    def _():
        pltpu.sync_copy(acc.at[pl.ds(s * DROWS, DROWS)],
                        s_ref.at[pl.ds(q * N_ATOMS + s * DROWS, DROWS)])

    @pl.when(s == NS - 1)
    def _():
        pltpu.sync_copy(acc.at[pl.ds(s * DROWS, DLAST)],
                        s_ref.at[pl.ds(q * N_ATOMS + s * DROWS, DLAST)])


def _sc_pass0_body(b2a_ref, b2aw_ref, b2revb_ref, lin_ref, t1_ref,
                   h_ref, s_ref, tgt_ref,
                   i1, i2, tg, tb, g1, lb, ob, zb, acc):
    c = lax.axis_index("c")
    s = lax.axis_index("s")
    base = s * BPS
    lane = jnp.arange(16, dtype=_i32)

    for p in range(2):
        q = 2 * c + p
        _zero_acc(zb, acc, s)
        plsc.subcore_barrier()

        @pl.loop(0, NCHUNK)
        def _(k):
            row0 = base + k * CHUNK
            pltpu.sync_copy(b2a_ref.at[pl.ds(row0, CHUNK)], i1)

            if p == 0:
                pltpu.sync_copy(b2revb_ref.at[pl.ds(row0, CHUNK)], i2)
                pltpu.sync_copy(b2aw_ref.at[i2], tb)

                @pl.loop(0, CHUNK // 16)
                def _(j):
                    res = jnp.zeros((16,), _i32)
                    for t in range(16):
                        res = jnp.where(lane == t, tb[j * 16 + t, :], res)
                    tg[pl.ds(j * 16, 16)] = res

                pltpu.sync_copy(tg, tgt_ref.at[pl.ds(c * N_BONDS + row0,
                                                     CHUNK)])
            else:
                pltpu.sync_copy(tgt_ref.at[pl.ds(c * N_BONDS + row0, CHUNK)],
                                tg)

            @pl.loop(0, CHUNK // 16)
            def _(j):
                i1[pl.ds(j * 16, 16)] = i1[pl.ds(j * 16, 16)] + q * N_ATOMS

            pltpu.sync_copy(t1_ref.at[i1], g1)
            pltpu.sync_copy(lin_ref.at[pl.ds(q * N_BONDS + row0, CHUNK)], lb)

            @pl.loop(0, CHUNK)
            def _(i):
                ob[i, :] = jnp.maximum(lb[i, :] + g1[i, :], 0.0)

            pltpu.sync_copy(ob, h_ref.at[pl.ds(q * N_BONDS + row0, CHUNK)])
            pltpu.sync_copy(ob, acc.at[tg], add=True)

        plsc.subcore_barrier()
        _dump_acc(acc, s_ref, q, s)
        if p == 0:
            plsc.subcore_barrier()


def _sc_depth_body(write_h, b2a_ref, b2revb_ref, tgt_in_ref, lin_ref, t1_ref,
                   t2_ref, *rest):
    if write_h:
        h_ref, s_ref = rest[0], rest[1]
        scratch = rest[2:]
    else:
        s_ref = rest[0]
        scratch = rest[1:]
    i1, i2, tg, g1, g2, lb, ob, zb, acc = scratch
    c = lax.axis_index("c")
    s = lax.axis_index("s")
    base = s * BPS

    for p in range(2):
        q = 2 * c + p
        _zero_acc(zb, acc, s)
        plsc.subcore_barrier()

        @pl.loop(0, NCHUNK)
        def _(k):
            row0 = base + k * CHUNK
            pltpu.sync_copy(b2a_ref.at[pl.ds(row0, CHUNK)], i1)
            pltpu.sync_copy(b2revb_ref.at[pl.ds(row0, CHUNK)], i2)
            pltpu.sync_copy(tgt_in_ref.at[pl.ds(c * N_BONDS + row0, CHUNK)],
                            tg)

            @pl.loop(0, CHUNK // 16)
            def _(j):
                i1[pl.ds(j * 16, 16)] = i1[pl.ds(j * 16, 16)] + q * N_ATOMS
                i2[pl.ds(j * 16, 16)] = i2[pl.ds(j * 16, 16)] + q * N_BONDS

            pltpu.sync_copy(t1_ref.at[i1], g1)
            pltpu.sync_copy(t2_ref.at[i2], g2)
            pltpu.sync_copy(lin_ref.at[pl.ds(q * N_BONDS + row0, CHUNK)], lb)

            @pl.loop(0, CHUNK)
            def _(i):
                ob[i, :] = jnp.maximum(lb[i, :] + g1[i, :] - g2[i, :], 0.0)

            if write_h:
                pltpu.sync_copy(ob, h_ref.at[pl.ds(q * N_BONDS + row0, CHUNK)])
            pltpu.sync_copy(ob, acc.at[tg], add=True)

        plsc.subcore_barrier()
        _dump_acc(acc, s_ref, q, s)
        if p == 0:
            plsc.subcore_barrier()


def _sc_pass0(b2a, b2a_wide, b2revb, B0q, A0q):
    return pl.kernel(
        _sc_pass0_body,
        out_type=[
            jax.ShapeDtypeStruct((NQ * N_BONDS, QW), _f32),
            jax.ShapeDtypeStruct((NQ * N_ATOMS, QW), _f32),
            jax.ShapeDtypeStruct((NC * N_BONDS,), _i32),
        ],
        mesh=_mesh,
        compiler_params=pltpu.CompilerParams(use_tc_tiling_on_sc=False),
        scratch_types=[
            pltpu.VMEM((CHUNK,), _i32),
            pltpu.VMEM((CHUNK,), _i32),
            pltpu.VMEM((CHUNK,), _i32),
            pltpu.VMEM((CHUNK, 16), _i32),
            pltpu.VMEM((CHUNK, QW), _f32),
            pltpu.VMEM((CHUNK, QW), _f32),
            pltpu.VMEM((CHUNK, QW), _f32),
            pltpu.VMEM((DROWS, QW), _f32),
            pltpu.VMEM_SHARED((N_ATOMS, QW), _f32),
        ],
    )(b2a, b2a_wide, b2revb, B0q, A0q)


def _sc_depth(b2a, b2revb, tgt, h0q, AM2q, Hhq, write_h):
    out_type = []
    if write_h:
        out_type.append(jax.ShapeDtypeStruct((NQ * N_BONDS, QW), _f32))
    out_type.append(jax.ShapeDtypeStruct((NQ * N_ATOMS, QW), _f32))
    outs = pl.kernel(
        functools.partial(_sc_depth_body, write_h),
        out_type=out_type,
        mesh=_mesh,
        compiler_params=pltpu.CompilerParams(use_tc_tiling_on_sc=False),
        scratch_types=[
            pltpu.VMEM((CHUNK,), _i32),
            pltpu.VMEM((CHUNK,), _i32),
            pltpu.VMEM((CHUNK,), _i32),
            pltpu.VMEM((CHUNK, QW), _f32),
            pltpu.VMEM((CHUNK, QW), _f32),
            pltpu.VMEM((CHUNK, QW), _f32),
            pltpu.VMEM((CHUNK, QW), _f32),
            pltpu.VMEM((DROWS, QW), _f32),
            pltpu.VMEM_SHARED((N_ATOMS, QW), _f32),
        ],
    )(b2a, b2revb, tgt, h0q, AM2q, Hhq)
    if write_h:
        return outs[0], outs[1]
    return None, (outs[0] if isinstance(outs, (tuple, list)) else outs)


def _qspec2(j, nb, blk):
    return pl.BlockSpec((blk, QW), lambda q, i: (j * nb + i, 0))


def _qspec1(j, nb, blk):
    return pl.BlockSpec((blk, QW), lambda i: (j * nb + i, 0))


def _split_w(wt):
    k = wt.shape[0]
    return wt.reshape(k, NQ, QW).transpose(1, 0, 2)


def _mm_split_body(x_ref, w_ref, o_ref):
    o_ref[...] = jnp.dot(x_ref[...], w_ref[0], preferred_element_type=_f32)


def _mm_split(x, wt, blk):
    n, k = x.shape
    nb = n // blk
    return pl.pallas_call(
        _mm_split_body,
        grid=(NQ, nb),
        in_specs=[
            pl.BlockSpec((blk, k), lambda q, i: (i, 0)),
            pl.BlockSpec((1, k, QW), lambda q, i: (q, 0, 0)),
        ],
        out_specs=pl.BlockSpec((blk, QW), lambda q, i: (q * nb + i, 0)),
        out_shape=jax.ShapeDtypeStruct((NQ * n, QW), _f32),
    )(x, _split_w(wt))


def _mm_stacked_body(x0_ref, x1_ref, x2_ref, x3_ref, w_ref, o_ref):
    x = jnp.concatenate(
        [x0_ref[...], x1_ref[...], x2_ref[...], x3_ref[...]], axis=1)
    o_ref[...] = jnp.dot(x, w_ref[0], preferred_element_type=_f32)


def _mm_stacked(xq, wt, n, blk):
    nb = n // blk
    return pl.pallas_call(
        _mm_stacked_body,
        grid=(NQ, nb),
        in_specs=[_qspec2(j, nb, blk) for j in range(NQ)]
        + [pl.BlockSpec((1, HIDDEN, QW), lambda q, i: (q, 0, 0))],
        out_specs=pl.BlockSpec((blk, QW), lambda q, i: (q * nb + i, 0)),
        out_shape=jax.ShapeDtypeStruct((NQ * n, QW), _f32),
    )(xq, xq, xq, xq, _split_w(wt))


def _final_body(x0_ref, x1_ref, x2_ref, x3_ref, fa_ref, wm_ref, wa_ref,
                b_ref, ids_ref, o_ref):
    i = pl.program_id(0)
    x = jnp.concatenate(
        [x0_ref[...], x1_ref[...], x2_ref[...], x3_ref[...]], axis=1)
    y = (jnp.dot(x, wm_ref[...], preferred_element_type=_f32)
         + jnp.dot(fa_ref[...], wa_ref[...], preferred_element_type=_f32)
         + b_ref[...])
    atom_h = jnp.maximum(y, 0.0)
    ids = ids_ref[0, 0, :]
    onehot = (lax.broadcasted_iota(_i32, (NMOLP, atom_h.shape[0]), 0)
              == ids[None, :]).astype(_f32)
    contrib = jnp.dot(onehot, atom_h, preferred_element_type=_f32)

    @pl.when(i == 0)
    def _():
        o_ref[...] = jnp.zeros_like(o_ref)

    o_ref[...] += contrib


def _final_readout(Sq, f_atoms, wm, wa, bias, mol_ids, blk=2000):
    nb = N_ATOMS // blk
    ids3 = mol_ids.reshape(nb, 1, blk)
    return pl.pallas_call(
        _final_body,
        grid=(nb,),
        in_specs=[_qspec1(j, nb, blk) for j in range(NQ)] + [
            pl.BlockSpec((blk, ATOM_FDIM), lambda i: (i, 0)),
            pl.BlockSpec((HIDDEN, HIDDEN), lambda i: (0, 0)),
            pl.BlockSpec((ATOM_FDIM, HIDDEN), lambda i: (0, 0)),
            pl.BlockSpec((1, HIDDEN), lambda i: (0, 0)),
            pl.BlockSpec((1, 1, blk), lambda i: (i, 0, 0)),
        ],
        out_specs=pl.BlockSpec((NMOLP, HIDDEN), lambda i: (0, 0)),
        out_shape=jax.ShapeDtypeStruct((NMOLP, HIDDEN), _f32),
    )(Sq, Sq, Sq, Sq, f_atoms, wm, wa, bias.reshape(1, -1), ids3)


def _head_body(mv_ref, r1w_ref, r1b_ref, r2w_ref, r2b_ref, o_ref):
    out = jnp.maximum(
        jnp.dot(mv_ref[...], r1w_ref[...], preferred_element_type=_f32)
        + r1b_ref[...], 0.0)
    o_ref[...] = (
        jnp.dot(out, r2w_ref[...], preferred_element_type=_f32)
        + r2b_ref[...])


def _head(mol_vecs, R1_t, R1_b, R2_t, R2_b):
    m = mol_vecs.shape[0]
    return pl.pallas_call(
        _head_body,
        out_shape=jax.ShapeDtypeStruct((m, TASKS), _f32),
    )(mol_vecs, R1_t, R1_b.reshape(1, -1), R2_t, R2_b.reshape(1, -1))


def kernel(f_atoms, f_bonds, b2a, b2revb, mol_ids, W_i, W_h, W_o_w, W_o_b,
           R1_w, R1_b, R2_w, R2_b):
    W_ia_t = W_i[:, :ATOM_FDIM].T
    W_ib_t = W_i[:, ATOM_FDIM:].T
    W_h_t = W_h.T
    W_oa_t = W_o_w[:, :ATOM_FDIM].T
    W_om_t = W_o_w[:, ATOM_FDIM:].T

    A0q = _mm_split(f_atoms, W_ia_t, blk=2000)
    B0q = _mm_split(f_bonds, W_ib_t, blk=8000)
    b2a_wide = jnp.broadcast_to(b2a[:, None], (N_BONDS, 16))

    h0q, Sq, tgt = _sc_pass0(b2a, b2a_wide, b2revb, B0q, A0q)

    hq = h0q
    for d in range(DEPTH):
        AM2q = _mm_stacked(Sq, W_h_t, N_ATOMS, blk=2000)
        Hhq = _mm_stacked(hq, W_h_t, N_BONDS, blk=8000)
        write_h = d < DEPTH - 1
        if write_h:
            hq, Sq = _sc_depth(b2a, b2revb, tgt, h0q, AM2q, Hhq, True)
        else:
            _, Sq = _sc_depth(b2a, b2revb, tgt, h0q, AM2q, Hhq, False)

    mol512 = _final_readout(Sq, f_atoms, W_om_t, W_oa_t, W_o_b, mol_ids)
    return _head(mol512[:N_MOLS], R1_w.T, R1_b, R2_w.T, R2_b)


# speedup vs baseline: 1.0003x; 1.0003x over previous
import functools

import jax
import jax.numpy as jnp
from jax import lax
from jax.experimental import pallas as pl
from jax.experimental.pallas import tpu as pltpu
from jax.experimental.pallas import tpu_sc as plsc

N_ATOMS = 50000
N_BONDS = 800000
ATOM_FDIM = 133
BOND_FDIM = 13
HIDDEN = 64
TASKS = 12
DEPTH = 3
N_MOLS = 500

NC = 2
NS = 16
NQ = 4
QW = HIDDEN // NQ
BPS = N_BONDS // NS
CHUNK = 80
NCHUNK = BPS // CHUNK
DROWS = 3128
DLAST = N_ATOMS - (NS - 1) * DROWS
NMOLP = 512

_mesh = plsc.VectorSubcoreMesh(core_axis_name="c", subcore_axis_name="s",
                               num_cores=2)
_f32 = jnp.float32
_i32 = jnp.int32


def _zero_acc(zb, acc, s):
    @pl.loop(0, DROWS)
    def _(i):
        zb[i, :] = jnp.zeros((QW,), _f32)

    @pl.when(s < NS - 1)
    def _():
        pltpu.sync_copy(zb, acc.at[pl.ds(s * DROWS, DROWS)])

    @pl.when(s == NS - 1)
    def _():
        pltpu.sync_copy(zb.at[pl.ds(0, DLAST)], acc.at[pl.ds(s * DROWS, DLAST)])


def _dump_acc(acc, s_ref, q, s):
    @pl.when(s < NS - 1)
    def _():
        pltpu.sync_copy(acc.at[pl.ds(s * DROWS, DROWS)],
                        s_ref.at[pl.ds(q * N_ATOMS + s * DROWS, DROWS)])

    @pl.when(s == NS - 1)
    def _():
        pltpu.sync_copy(acc.at[pl.ds(s * DROWS, DLAST)],
                        s_ref.at[pl.ds(q * N_ATOMS + s * DROWS, DLAST)])


def _sc_pass0_body(b2a_ref, b2aw_ref, b2revb_ref, lin_ref, t1_ref,
                   h_ref, s_ref, tgt_ref,
                   i1, i2, tg, tb, g1, lb, ob, zb, acc):
    c = lax.axis_index("c")
    s = lax.axis_index("s")
    base = s * BPS
    lane = jnp.arange(16, dtype=_i32)

    for p in range(2):
        q = 2 * c + p
        _zero_acc(zb, acc, s)
        plsc.subcore_barrier()

        @pl.loop(0, NCHUNK)
        def _(k):
            row0 = base + k * CHUNK
            pltpu.sync_copy(b2a_ref.at[pl.ds(row0, CHUNK)], i1)

            if p == 0:
                pltpu.sync_copy(b2revb_ref.at[pl.ds(row0, CHUNK)], i2)
                pltpu.sync_copy(b2aw_ref.at[i2], tb)

                @pl.loop(0, CHUNK // 16)
                def _(j):
                    res = jnp.zeros((16,), _i32)
                    for t in range(16):
                        res = jnp.where(lane == t, tb[j * 16 + t, :], res)
                    tg[pl.ds(j * 16, 16)] = res

                pltpu.sync_copy(tg, tgt_ref.at[pl.ds(c * N_BONDS + row0,
                                                     CHUNK)])
            else:
                pltpu.sync_copy(tgt_ref.at[pl.ds(c * N_BONDS + row0, CHUNK)],
                                tg)

            @pl.loop(0, CHUNK // 16)
            def _(j):
                i1[pl.ds(j * 16, 16)] = i1[pl.ds(j * 16, 16)] + q * N_ATOMS

            pltpu.sync_copy(t1_ref.at[i1], g1)
            pltpu.sync_copy(lin_ref.at[pl.ds(q * N_BONDS + row0, CHUNK)], lb)

            @pl.loop(0, CHUNK)
            def _(i):
                ob[i, :] = jnp.maximum(lb[i, :] + g1[i, :], 0.0)

            pltpu.sync_copy(ob, h_ref.at[pl.ds(q * N_BONDS + row0, CHUNK)])
            pltpu.sync_copy(ob, acc.at[tg], add=True)

        plsc.subcore_barrier()
        _dump_acc(acc, s_ref, q, s)
        if p == 0:
            plsc.subcore_barrier()


def _sc_depth_body(write_h, b2a_ref, b2revb_ref, tgt_in_ref, lin_ref, t1_ref,
                   t2_ref, *rest):
    if write_h:
        h_ref, s_ref = rest[0], rest[1]
        scratch = rest[2:]
    else:
        s_ref = rest[0]
        scratch = rest[1:]
    i1, i2, tg, g1, g2, lb, ob, zb, acc = scratch
    c = lax.axis_index("c")
    s = lax.axis_index("s")
    base = s * BPS

    for p in range(2):
        q = 2 * c + p
        _zero_acc(zb, acc, s)
        plsc.subcore_barrier()

        @pl.loop(0, NCHUNK)
        def _(k):
            row0 = base + k * CHUNK
            pltpu.sync_copy(b2a_ref.at[pl.ds(row0, CHUNK)], i1)
            pltpu.sync_copy(b2revb_ref.at[pl.ds(row0, CHUNK)], i2)
            pltpu.sync_copy(tgt_in_ref.at[pl.ds(c * N_BONDS + row0, CHUNK)],
                            tg)

            @pl.loop(0, CHUNK // 16)
            def _(j):
                i1[pl.ds(j * 16, 16)] = i1[pl.ds(j * 16, 16)] + q * N_ATOMS
                i2[pl.ds(j * 16, 16)] = i2[pl.ds(j * 16, 16)] + q * N_BONDS

            pltpu.sync_copy(t1_ref.at[i1], g1)
            pltpu.sync_copy(t2_ref.at[i2], g2)
            pltpu.sync_copy(lin_ref.at[pl.ds(q * N_BONDS + row0, CHUNK)], lb)

            @pl.loop(0, CHUNK)
            def _(i):
                ob[i, :] = jnp.maximum(lb[i, :] + g1[i, :] - g2[i, :], 0.0)

            if write_h:
                pltpu.sync_copy(ob, h_ref.at[pl.ds(q * N_BONDS + row0, CHUNK)])
            pltpu.sync_copy(ob, acc.at[tg], add=True)

        plsc.subcore_barrier()
        _dump_acc(acc, s_ref, q, s)
        if p == 0:
            plsc.subcore_barrier()


def _sc_pass0(b2a, b2a_wide, b2revb, B0q, A0q):
    return pl.kernel(
        _sc_pass0_body,
        out_type=[
            jax.ShapeDtypeStruct((NQ * N_BONDS, QW), _f32),
            jax.ShapeDtypeStruct((NQ * N_ATOMS, QW), _f32),
            jax.ShapeDtypeStruct((NC * N_BONDS,), _i32),
        ],
        mesh=_mesh,
        compiler_params=pltpu.CompilerParams(use_tc_tiling_on_sc=False),
        scratch_types=[
            pltpu.VMEM((CHUNK,), _i32),
            pltpu.VMEM((CHUNK,), _i32),
            pltpu.VMEM((CHUNK,), _i32),
            pltpu.VMEM((CHUNK, 16), _i32),
            pltpu.VMEM((CHUNK, QW), _f32),
            pltpu.VMEM((CHUNK, QW), _f32),
            pltpu.VMEM((CHUNK, QW), _f32),
            pltpu.VMEM((DROWS, QW), _f32),
            pltpu.VMEM_SHARED((N_ATOMS, QW), _f32),
        ],
    )(b2a, b2a_wide, b2revb, B0q, A0q)


def _sc_depth(b2a, b2revb, tgt, h0q, AM2q, Hhq, write_h):
    out_type = []
    if write_h:
        out_type.append(jax.ShapeDtypeStruct((NQ * N_BONDS, QW), _f32))
    out_type.append(jax.ShapeDtypeStruct((NQ * N_ATOMS, QW), _f32))
    outs = pl.kernel(
        functools.partial(_sc_depth_body, write_h),
        out_type=out_type,
        mesh=_mesh,
        compiler_params=pltpu.CompilerParams(use_tc_tiling_on_sc=False),
        scratch_types=[
            pltpu.VMEM((CHUNK,), _i32),
            pltpu.VMEM((CHUNK,), _i32),
            pltpu.VMEM((CHUNK,), _i32),
            pltpu.VMEM((CHUNK, QW), _f32),
            pltpu.VMEM((CHUNK, QW), _f32),
            pltpu.VMEM((CHUNK, QW), _f32),
            pltpu.VMEM((CHUNK, QW), _f32),
            pltpu.VMEM((DROWS, QW), _f32),
            pltpu.VMEM_SHARED((N_ATOMS, QW), _f32),
        ],
    )(b2a, b2revb, tgt, h0q, AM2q, Hhq)
    if write_h:
        return outs[0], outs[1]
    return None, (outs[0] if isinstance(outs, (tuple, list)) else outs)


def _qspec2(j, nb, blk):
    return pl.BlockSpec((blk, QW), lambda q, i: (j * nb + i, 0))


def _qspec1(j, nb, blk):
    return pl.BlockSpec((blk, QW), lambda i: (j * nb + i, 0))


def _split_w(wt):
    k = wt.shape[0]
    return wt.reshape(k, NQ, QW).transpose(1, 0, 2)


def _mm_split_body(x_ref, w_ref, o_ref):
    o_ref[...] = jnp.dot(x_ref[...], w_ref[0], preferred_element_type=_f32)


def _mm_split(x, wt, blk):
    n, k = x.shape
    nb = n // blk
    return pl.pallas_call(
        _mm_split_body,
        grid=(NQ, nb),
        in_specs=[
            pl.BlockSpec((blk, k), lambda q, i: (i, 0)),
            pl.BlockSpec((1, k, QW), lambda q, i: (q, 0, 0)),
        ],
        out_specs=pl.BlockSpec((blk, QW), lambda q, i: (q * nb + i, 0)),
        out_shape=jax.ShapeDtypeStruct((NQ * n, QW), _f32),
    )(x, _split_w(wt))


def _mm_stacked_body(x0_ref, x1_ref, x2_ref, x3_ref, w_ref, o_ref):
    x = jnp.concatenate(
        [x0_ref[...], x1_ref[...], x2_ref[...], x3_ref[...]], axis=1)
    o_ref[...] = jnp.dot(x, w_ref[0], preferred_element_type=_f32)


def _mm_stacked(xq, wt, n, blk):
    nb = n // blk
    return pl.pallas_call(
        _mm_stacked_body,
        grid=(NQ, nb),
        in_specs=[_qspec2(j, nb, blk) for j in range(NQ)]
        + [pl.BlockSpec((1, HIDDEN, QW), lambda q, i: (q, 0, 0))],
        out_specs=pl.BlockSpec((blk, QW), lambda q, i: (q * nb + i, 0)),
        out_shape=jax.ShapeDtypeStruct((NQ * n, QW), _f32),
    )(xq, xq, xq, xq, _split_w(wt))


def _final_body(x0_ref, x1_ref, x2_ref, x3_ref, fa_ref, wm_ref, wa_ref,
                b_ref, ids_ref, o_ref):
    i = pl.program_id(0)
    x = jnp.concatenate(
        [x0_ref[...], x1_ref[...], x2_ref[...], x3_ref[...]], axis=1)
    y = (jnp.dot(x, wm_ref[...], preferred_element_type=_f32)
         + jnp.dot(fa_ref[...], wa_ref[...], preferred_element_type=_f32)
         + b_ref[...])
    atom_h = jnp.maximum(y, 0.0)
    ids = ids_ref[0, 0, :]
    onehot = (lax.broadcasted_iota(_i32, (NMOLP, atom_h.shape[0]), 0)
              == ids[None, :]).astype(_f32)
    contrib = jnp.dot(onehot, atom_h, preferred_element_type=_f32)

    @pl.when(i == 0)
    def _():
        o_ref[...] = jnp.zeros_like(o_ref)

    o_ref[...] += contrib


def _final_readout(Sq, f_atoms, wm, wa, bias, mol_ids, blk=2000):
    nb = N_ATOMS // blk
    ids3 = mol_ids.reshape(nb, 1, blk)
    return pl.pallas_call(
        _final_body,
        grid=(nb,),
        in_specs=[_qspec1(j, nb, blk) for j in range(NQ)] + [
            pl.BlockSpec((blk, ATOM_FDIM), lambda i: (i, 0)),
            pl.BlockSpec((HIDDEN, HIDDEN), lambda i: (0, 0)),
            pl.BlockSpec((ATOM_FDIM, HIDDEN), lambda i: (0, 0)),
            pl.BlockSpec((1, HIDDEN), lambda i: (0, 0)),
            pl.BlockSpec((1, 1, blk), lambda i: (i, 0, 0)),
        ],
        out_specs=pl.BlockSpec((NMOLP, HIDDEN), lambda i: (0, 0)),
        out_shape=jax.ShapeDtypeStruct((NMOLP, HIDDEN), _f32),
    )(Sq, Sq, Sq, Sq, f_atoms, wm, wa, bias.reshape(1, -1), ids3)


def _head_body(mv_ref, r1w_ref, r1b_ref, r2w_ref, r2b_ref, o_ref):
    out = jnp.maximum(
        jnp.dot(mv_ref[...], r1w_ref[...], preferred_element_type=_f32)
        + r1b_ref[...], 0.0)
    o_ref[...] = (
        jnp.dot(out, r2w_ref[...], preferred_element_type=_f32)
        + r2b_ref[...])


def _head(mol_vecs, R1_t, R1_b, R2_t, R2_b):
    m = mol_vecs.shape[0]
    return pl.pallas_call(
        _head_body,
        out_shape=jax.ShapeDtypeStruct((m, TASKS), _f32),
    )(mol_vecs, R1_t, R1_b.reshape(1, -1), R2_t, R2_b.reshape(1, -1))


def kernel(f_atoms, f_bonds, b2a, b2revb, mol_ids, W_i, W_h, W_o_w, W_o_b,
           R1_w, R1_b, R2_w, R2_b):
    W_ia_t = W_i[:, :ATOM_FDIM].T
    W_ib_t = W_i[:, ATOM_FDIM:].T
    W_h_t = W_h.T
    W_oa_t = W_o_w[:, :ATOM_FDIM].T
    W_om_t = W_o_w[:, ATOM_FDIM:].T

    A0q = _mm_split(f_atoms, W_ia_t, blk=2000)
    B0q = _mm_split(f_bonds, W_ib_t, blk=8000)
    b2a_wide = jnp.broadcast_to(b2a[:, None], (N_BONDS, 16))

    h0q, Sq, tgt = _sc_pass0(b2a, b2a_wide, b2revb, B0q, A0q)

    hq = h0q
    for d in range(DEPTH):
        AM2q = _mm_stacked(Sq, W_h_t, N_ATOMS, blk=2000)
        Hhq = _mm_stacked(hq, W_h_t, N_BONDS, blk=8000)
        write_h = d < DEPTH - 1
        if write_h:
            hq, Sq = _sc_depth(b2a, b2revb, tgt, h0q, AM2q, Hhq, True)
        else:
            _, Sq = _sc_depth(b2a, b2revb, tgt, h0q, AM2q, Hhq, False)

    mol512 = _final_readout(Sq, f_atoms, W_om_t, W_oa_t, W_o_b, mol_ids)
    return _head(mol512[:N_MOLS], R1_w.T, R1_b, R2_w.T, R2_b)


# speedup vs baseline: 1.3314x; 1.3310x over previous
import functools

import jax
import jax.numpy as jnp
from jax import lax
from jax.experimental import pallas as pl
from jax.experimental.pallas import tpu as pltpu
from jax.experimental.pallas import tpu_sc as plsc

N_ATOMS = 50000
N_BONDS = 800000
ATOM_FDIM = 133
BOND_FDIM = 13
HIDDEN = 64
TASKS = 12
DEPTH = 3
N_MOLS = 500

NC = 2
NS = 16
NQ = 4
QW = HIDDEN // NQ
BPS = N_BONDS // NS
CHUNK = 80
NCHUNK = BPS // CHUNK
DROWS = 3128
DLAST = N_ATOMS - (NS - 1) * DROWS
NMOLP = 512

_mesh = plsc.VectorSubcoreMesh(core_axis_name="c", subcore_axis_name="s",
                               num_cores=2)
_f32 = jnp.float32
_i32 = jnp.int32


def _zero_acc(zb, acc, s):
    @pl.loop(0, DROWS)
    def _(i):
        zb[i, :] = jnp.zeros((QW,), _f32)

    @pl.when(s < NS - 1)
    def _():
        pltpu.sync_copy(zb, acc.at[pl.ds(s * DROWS, DROWS)])

    @pl.when(s == NS - 1)
    def _():
        pltpu.sync_copy(zb.at[pl.ds(0, DLAST)], acc.at[pl.ds(s * DROWS, DLAST)])


def _dump_acc(acc, s_ref, q, s):
    @pl.when(s < NS - 1)
    def _():
        pltpu.sync_copy(acc.at[pl.ds(s * DROWS, DROWS)],
                        s_ref.at[pl.ds(q * N_ATOMS + s * DROWS, DROWS)])

    @pl.when(s == NS - 1)
    def _():
        pltpu.sync_copy(acc.at[pl.ds(s * DROWS, DLAST)],
                        s_ref.at[pl.ds(q * N_ATOMS + s * DROWS, DLAST)])


def _sc_pass0_body(b2a_ref, b2aw_ref, b2revb_ref, lin_ref, t1_ref,
                   h_ref, s_ref, tgt_ref,
                   i1, i2, tg, tb, g1, lb, ob, zb, acc, sem1, sem2, sem3):
    c = lax.axis_index("c")
    s = lax.axis_index("s")
    base = s * BPS
    lane = jnp.arange(16, dtype=_i32)

    for p in range(2):
        q = 2 * c + p
        _zero_acc(zb, acc, s)
        plsc.subcore_barrier()

        @pl.loop(0, NCHUNK)
        def _(k):
            row0 = base + k * CHUNK
            a1 = pltpu.async_copy(b2a_ref.at[pl.ds(row0, CHUNK)], i1, sem1)

            if p == 0:
                a2 = pltpu.async_copy(
                    b2revb_ref.at[pl.ds(row0, CHUNK)], i2, sem1)
                a1.wait()
                a2.wait()
                tbc = pltpu.async_copy(b2aw_ref.at[i2], tb, sem2)

                @pl.loop(0, CHUNK // 16)
                def _(j):
                    i1[pl.ds(j * 16, 16)] = i1[pl.ds(j * 16, 16)] + q * N_ATOMS

                tbc.wait()

                @pl.loop(0, CHUNK // 16)
                def _(j):
                    res = jnp.zeros((16,), _i32)
                    for t in range(16):
                        res = jnp.where(lane == t, tb[j * 16 + t, :], res)
                    tg[pl.ds(j * 16, 16)] = res

                tgw = pltpu.async_copy(
                    tg, tgt_ref.at[pl.ds(c * N_BONDS + row0, CHUNK)], sem3)
            else:
                a3 = pltpu.async_copy(
                    tgt_ref.at[pl.ds(c * N_BONDS + row0, CHUNK)], tg, sem1)
                a1.wait()
                a3.wait()

                @pl.loop(0, CHUNK // 16)
                def _(j):
                    i1[pl.ds(j * 16, 16)] = i1[pl.ds(j * 16, 16)] + q * N_ATOMS

            g1c = pltpu.async_copy(t1_ref.at[i1], g1, sem2)
            lbc = pltpu.async_copy(
                lin_ref.at[pl.ds(q * N_BONDS + row0, CHUNK)], lb, sem2)
            g1c.wait()
            lbc.wait()

            @pl.loop(0, CHUNK)
            def _(i):
                ob[i, :] = jnp.maximum(lb[i, :] + g1[i, :], 0.0)

            hw = pltpu.async_copy(
                ob, h_ref.at[pl.ds(q * N_BONDS + row0, CHUNK)], sem3)
            pltpu.sync_copy(ob, acc.at[tg], add=True)
            hw.wait()
            if p == 0:
                tgw.wait()

        plsc.subcore_barrier()
        _dump_acc(acc, s_ref, q, s)
        if p == 0:
            plsc.subcore_barrier()


def _sc_depth_body(write_h, b2a_ref, b2revb_ref, tgt_in_ref, lin_ref, t1_ref,
                   t2_ref, *rest):
    if write_h:
        h_ref, s_ref = rest[0], rest[1]
        scratch = rest[2:]
    else:
        s_ref = rest[0]
        scratch = rest[1:]
    i1, i2, tg, g1, g2, lb, ob, zb, acc, sem1, sem2, sem3 = scratch
    c = lax.axis_index("c")
    s = lax.axis_index("s")
    base = s * BPS

    for p in range(2):
        q = 2 * c + p
        _zero_acc(zb, acc, s)
        plsc.subcore_barrier()

        @pl.loop(0, NCHUNK)
        def _(k):
            row0 = base + k * CHUNK
            a1 = pltpu.async_copy(b2a_ref.at[pl.ds(row0, CHUNK)], i1, sem1)
            a2 = pltpu.async_copy(b2revb_ref.at[pl.ds(row0, CHUNK)], i2, sem1)
            a3 = pltpu.async_copy(
                tgt_in_ref.at[pl.ds(c * N_BONDS + row0, CHUNK)], tg, sem1)
            a1.wait()
            a2.wait()
            a3.wait()

            @pl.loop(0, CHUNK // 16)
            def _(j):
                i1[pl.ds(j * 16, 16)] = i1[pl.ds(j * 16, 16)] + q * N_ATOMS
                i2[pl.ds(j * 16, 16)] = i2[pl.ds(j * 16, 16)] + q * N_BONDS

            g1c = pltpu.async_copy(t1_ref.at[i1], g1, sem2)
            g2c = pltpu.async_copy(t2_ref.at[i2], g2, sem2)
            lbc = pltpu.async_copy(
                lin_ref.at[pl.ds(q * N_BONDS + row0, CHUNK)], lb, sem2)
            g1c.wait()
            g2c.wait()
            lbc.wait()

            @pl.loop(0, CHUNK)
            def _(i):
                ob[i, :] = jnp.maximum(lb[i, :] + g1[i, :] - g2[i, :], 0.0)

            if write_h:
                hw = pltpu.async_copy(
                    ob, h_ref.at[pl.ds(q * N_BONDS + row0, CHUNK)], sem3)
            pltpu.sync_copy(ob, acc.at[tg], add=True)
            if write_h:
                hw.wait()

        plsc.subcore_barrier()
        _dump_acc(acc, s_ref, q, s)
        if p == 0:
            plsc.subcore_barrier()


def _sc_pass0(b2a, b2a_wide, b2revb, B0q, A0q):
    return pl.kernel(
        _sc_pass0_body,
        out_type=[
            jax.ShapeDtypeStruct((NQ * N_BONDS, QW), _f32),
            jax.ShapeDtypeStruct((NQ * N_ATOMS, QW), _f32),
            jax.ShapeDtypeStruct((NC * N_BONDS,), _i32),
        ],
        mesh=_mesh,
        compiler_params=pltpu.CompilerParams(use_tc_tiling_on_sc=False),
        scratch_types=[
            pltpu.VMEM((CHUNK,), _i32),
            pltpu.VMEM((CHUNK,), _i32),
            pltpu.VMEM((CHUNK,), _i32),
            pltpu.VMEM((CHUNK, 16), _i32),
            pltpu.VMEM((CHUNK, QW), _f32),
            pltpu.VMEM((CHUNK, QW), _f32),
            pltpu.VMEM((CHUNK, QW), _f32),
            pltpu.VMEM((DROWS, QW), _f32),
            pltpu.VMEM_SHARED((N_ATOMS, QW), _f32),
            pltpu.SemaphoreType.DMA,
            pltpu.SemaphoreType.DMA,
            pltpu.SemaphoreType.DMA,
        ],
    )(b2a, b2a_wide, b2revb, B0q, A0q)


def _sc_depth(b2a, b2revb, tgt, h0q, AM2q, Hhq, write_h):
    out_type = []
    if write_h:
        out_type.append(jax.ShapeDtypeStruct((NQ * N_BONDS, QW), _f32))
    out_type.append(jax.ShapeDtypeStruct((NQ * N_ATOMS, QW), _f32))
    outs = pl.kernel(
        functools.partial(_sc_depth_body, write_h),
        out_type=out_type,
        mesh=_mesh,
        compiler_params=pltpu.CompilerParams(use_tc_tiling_on_sc=False),
        scratch_types=[
            pltpu.VMEM((CHUNK,), _i32),
            pltpu.VMEM((CHUNK,), _i32),
            pltpu.VMEM((CHUNK,), _i32),
            pltpu.VMEM((CHUNK, QW), _f32),
            pltpu.VMEM((CHUNK, QW), _f32),
            pltpu.VMEM((CHUNK, QW), _f32),
            pltpu.VMEM((CHUNK, QW), _f32),
            pltpu.VMEM((DROWS, QW), _f32),
            pltpu.VMEM_SHARED((N_ATOMS, QW), _f32),
            pltpu.SemaphoreType.DMA,
            pltpu.SemaphoreType.DMA,
            pltpu.SemaphoreType.DMA,
        ],
    )(b2a, b2revb, tgt, h0q, AM2q, Hhq)
    if write_h:
        return outs[0], outs[1]
    return None, (outs[0] if isinstance(outs, (tuple, list)) else outs)


def _qspec2(j, nb, blk):
    return pl.BlockSpec((blk, QW), lambda q, i: (j * nb + i, 0))


def _qspec1(j, nb, blk):
    return pl.BlockSpec((blk, QW), lambda i: (j * nb + i, 0))


def _split_w(wt):
    k = wt.shape[0]
    return wt.reshape(k, NQ, QW).transpose(1, 0, 2)


def _mm_split_body(x_ref, w_ref, o_ref):
    o_ref[...] = jnp.dot(x_ref[...], w_ref[0], preferred_element_type=_f32)


def _mm_split(x, wt, blk):
    n, k = x.shape
    nb = n // blk
    return pl.pallas_call(
        _mm_split_body,
        grid=(NQ, nb),
        in_specs=[
            pl.BlockSpec((blk, k), lambda q, i: (i, 0)),
            pl.BlockSpec((1, k, QW), lambda q, i: (q, 0, 0)),
        ],
        out_specs=pl.BlockSpec((blk, QW), lambda q, i: (q * nb + i, 0)),
        out_shape=jax.ShapeDtypeStruct((NQ * n, QW), _f32),
    )(x, _split_w(wt))


def _mm_stacked_body(x0_ref, x1_ref, x2_ref, x3_ref, w_ref, o_ref):
    x = jnp.concatenate(
        [x0_ref[...], x1_ref[...], x2_ref[...], x3_ref[...]], axis=1)
    o_ref[...] = jnp.dot(x, w_ref[0], preferred_element_type=_f32)


def _mm_stacked(xq, wt, n, blk):
    nb = n // blk
    return pl.pallas_call(
        _mm_stacked_body,
        grid=(NQ, nb),
        in_specs=[_qspec2(j, nb, blk) for j in range(NQ)]
        + [pl.BlockSpec((1, HIDDEN, QW), lambda q, i: (q, 0, 0))],
        out_specs=pl.BlockSpec((blk, QW), lambda q, i: (q * nb + i, 0)),
        out_shape=jax.ShapeDtypeStruct((NQ * n, QW), _f32),
    )(xq, xq, xq, xq, _split_w(wt))


def _final_body(x0_ref, x1_ref, x2_ref, x3_ref, fa_ref, wm_ref, wa_ref,
                b_ref, ids_ref, o_ref):
    i = pl.program_id(0)
    x = jnp.concatenate(
        [x0_ref[...], x1_ref[...], x2_ref[...], x3_ref[...]], axis=1)
    y = (jnp.dot(x, wm_ref[...], preferred_element_type=_f32)
         + jnp.dot(fa_ref[...], wa_ref[...], preferred_element_type=_f32)
         + b_ref[...])
    atom_h = jnp.maximum(y, 0.0)
    ids = ids_ref[0, 0, :]
    onehot = (lax.broadcasted_iota(_i32, (NMOLP, atom_h.shape[0]), 0)
              == ids[None, :]).astype(_f32)
    contrib = jnp.dot(onehot, atom_h, preferred_element_type=_f32)

    @pl.when(i == 0)
    def _():
        o_ref[...] = jnp.zeros_like(o_ref)

    o_ref[...] += contrib


def _final_readout(Sq, f_atoms, wm, wa, bias, mol_ids, blk=2000):
    nb = N_ATOMS // blk
    ids3 = mol_ids.reshape(nb, 1, blk)
    return pl.pallas_call(
        _final_body,
        grid=(nb,),
        in_specs=[_qspec1(j, nb, blk) for j in range(NQ)] + [
            pl.BlockSpec((blk, ATOM_FDIM), lambda i: (i, 0)),
            pl.BlockSpec((HIDDEN, HIDDEN), lambda i: (0, 0)),
            pl.BlockSpec((ATOM_FDIM, HIDDEN), lambda i: (0, 0)),
            pl.BlockSpec((1, HIDDEN), lambda i: (0, 0)),
            pl.BlockSpec((1, 1, blk), lambda i: (i, 0, 0)),
        ],
        out_specs=pl.BlockSpec((NMOLP, HIDDEN), lambda i: (0, 0)),
        out_shape=jax.ShapeDtypeStruct((NMOLP, HIDDEN), _f32),
    )(Sq, Sq, Sq, Sq, f_atoms, wm, wa, bias.reshape(1, -1), ids3)


def _head_body(mv_ref, r1w_ref, r1b_ref, r2w_ref, r2b_ref, o_ref):
    out = jnp.maximum(
        jnp.dot(mv_ref[...], r1w_ref[...], preferred_element_type=_f32)
        + r1b_ref[...], 0.0)
    o_ref[...] = (
        jnp.dot(out, r2w_ref[...], preferred_element_type=_f32)
        + r2b_ref[...])


def _head(mol_vecs, R1_t, R1_b, R2_t, R2_b):
    m = mol_vecs.shape[0]
    return pl.pallas_call(
        _head_body,
        out_shape=jax.ShapeDtypeStruct((m, TASKS), _f32),
    )(mol_vecs, R1_t, R1_b.reshape(1, -1), R2_t, R2_b.reshape(1, -1))


def kernel(f_atoms, f_bonds, b2a, b2revb, mol_ids, W_i, W_h, W_o_w, W_o_b,
           R1_w, R1_b, R2_w, R2_b):
    W_ia_t = W_i[:, :ATOM_FDIM].T
    W_ib_t = W_i[:, ATOM_FDIM:].T
    W_h_t = W_h.T
    W_oa_t = W_o_w[:, :ATOM_FDIM].T
    W_om_t = W_o_w[:, ATOM_FDIM:].T

    A0q = _mm_split(f_atoms, W_ia_t, blk=2000)
    B0q = _mm_split(f_bonds, W_ib_t, blk=8000)
    b2a_wide = jnp.broadcast_to(b2a[:, None], (N_BONDS, 16))

    h0q, Sq, tgt = _sc_pass0(b2a, b2a_wide, b2revb, B0q, A0q)

    hq = h0q
    for d in range(DEPTH):
        AM2q = _mm_stacked(Sq, W_h_t, N_ATOMS, blk=2000)
        Hhq = _mm_stacked(hq, W_h_t, N_BONDS, blk=8000)
        write_h = d < DEPTH - 1
        if write_h:
            hq, Sq = _sc_depth(b2a, b2revb, tgt, h0q, AM2q, Hhq, True)
        else:
            _, Sq = _sc_depth(b2a, b2revb, tgt, h0q, AM2q, Hhq, False)

    mol512 = _final_readout(Sq, f_atoms, W_om_t, W_oa_t, W_o_b, mol_ids)
    return _head(mol512[:N_MOLS], R1_w.T, R1_b, R2_w.T, R2_b)


# speedup vs baseline: 1.5798x; 1.1866x over previous
import functools

import jax
import jax.numpy as jnp
from jax import lax
from jax.experimental import pallas as pl
from jax.experimental.pallas import tpu as pltpu
from jax.experimental.pallas import tpu_sc as plsc

N_ATOMS = 50000
N_BONDS = 800000
ATOM_FDIM = 133
BOND_FDIM = 13
HIDDEN = 64
TASKS = 12
DEPTH = 3
N_MOLS = 500

NC = 2
NS = 16
NQ = 4
QW = HIDDEN // NQ
BPS = N_BONDS // NS
CHUNK = 80
NCHUNK = BPS // CHUNK
DROWS = 3128
DLAST = N_ATOMS - (NS - 1) * DROWS
NMOLP = 512

_mesh = plsc.VectorSubcoreMesh(core_axis_name="c", subcore_axis_name="s",
                               num_cores=2)
_f32 = jnp.float32
_i32 = jnp.int32


def _zero_acc(zb, acc, s):
    @pl.loop(0, DROWS)
    def _(i):
        zb[i, :] = jnp.zeros((QW,), _f32)

    @pl.when(s < NS - 1)
    def _():
        pltpu.sync_copy(zb, acc.at[pl.ds(s * DROWS, DROWS)])

    @pl.when(s == NS - 1)
    def _():
        pltpu.sync_copy(zb.at[pl.ds(0, DLAST)], acc.at[pl.ds(s * DROWS, DLAST)])


def _dump_acc(acc, s_ref, q, s):
    @pl.when(s < NS - 1)
    def _():
        pltpu.sync_copy(acc.at[pl.ds(s * DROWS, DROWS)],
                        s_ref.at[pl.ds(q * N_ATOMS + s * DROWS, DROWS)])

    @pl.when(s == NS - 1)
    def _():
        pltpu.sync_copy(acc.at[pl.ds(s * DROWS, DLAST)],
                        s_ref.at[pl.ds(q * N_ATOMS + s * DROWS, DLAST)])


def _sc_pass0_body(b2a_ref, b2aw_ref, b2revb_ref, lin_ref, t1_ref,
                   h_ref, s_ref, tgt_ref,
                   i1, i2, tg, tb, g1, lb, ob, zb, acc, sem1, sem2, sem3):
    c = lax.axis_index("c")
    s = lax.axis_index("s")
    base = s * BPS
    lane = jnp.arange(16, dtype=_i32)

    for p in range(2):
        q = 2 * c + p
        _zero_acc(zb, acc, s)
        plsc.subcore_barrier()

        @pl.loop(0, NCHUNK)
        def _(k):
            row0 = base + k * CHUNK
            a1 = pltpu.async_copy(b2a_ref.at[pl.ds(row0, CHUNK)], i1, sem1)

            if p == 0:
                a2 = pltpu.async_copy(
                    b2revb_ref.at[pl.ds(row0, CHUNK)], i2, sem1)
                a1.wait()
                a2.wait()
                tbc = pltpu.async_copy(b2aw_ref.at[i2], tb, sem2)

                @pl.loop(0, CHUNK // 16)
                def _(j):
                    i1[pl.ds(j * 16, 16)] = i1[pl.ds(j * 16, 16)] + q * N_ATOMS

                tbc.wait()

                @pl.loop(0, CHUNK // 16)
                def _(j):
                    res = jnp.zeros((16,), _i32)
                    for t in range(16):
                        res = jnp.where(lane == t, tb[j * 16 + t, :], res)
                    tg[pl.ds(j * 16, 16)] = res

                tgw = pltpu.async_copy(
                    tg, tgt_ref.at[pl.ds(c * N_BONDS + row0, CHUNK)], sem3)
            else:
                a3 = pltpu.async_copy(
                    tgt_ref.at[pl.ds(c * N_BONDS + row0, CHUNK)], tg, sem1)
                a1.wait()
                a3.wait()

                @pl.loop(0, CHUNK // 16)
                def _(j):
                    i1[pl.ds(j * 16, 16)] = i1[pl.ds(j * 16, 16)] + q * N_ATOMS

            g1c = pltpu.async_copy(t1_ref.at[i1], g1, sem2)
            lbc = pltpu.async_copy(
                lin_ref.at[pl.ds(q * N_BONDS + row0, CHUNK)], lb, sem2)
            g1c.wait()
            lbc.wait()

            @pl.loop(0, CHUNK)
            def _(i):
                ob[i, :] = jnp.maximum(lb[i, :] + g1[i, :], 0.0)

            hw = pltpu.async_copy(
                ob, h_ref.at[pl.ds(q * N_BONDS + row0, CHUNK)], sem3)
            pltpu.sync_copy(ob, acc.at[tg], add=True)
            hw.wait()
            if p == 0:
                tgw.wait()

        plsc.subcore_barrier()
        _dump_acc(acc, s_ref, q, s)
        if p == 0:
            plsc.subcore_barrier()


NB = 5


def _sc_depth_body(write_h, b2aq_ref, b2rq_ref, tgt_in_ref, lin_ref, t1_ref,
                   t2_ref, *rest):
    if write_h:
        h_ref, s_ref = rest[0], rest[1]
        scratch = rest[2:]
    else:
        s_ref = rest[0]
        scratch = rest[1:]
    i1a, i2a = scratch[0], scratch[1]
    tgs = list(scratch[2:2 + NB])
    g1a, g2a, lba, oba, zb, acc = scratch[2 + NB:8 + NB]
    sems = scratch[8 + NB:]
    semI, semG, semS, semW = (sems[0:NB], sems[NB:2 * NB],
                              sems[2 * NB:3 * NB], sems[3 * NB:4 * NB])
    c = lax.axis_index("c")
    s = lax.axis_index("s")
    base = s * BPS

    for p in range(2):
        q = 2 * c + p
        qb = q * N_BONDS

        _zero_acc(zb, acc, s)
        plsc.subcore_barrier()

        def issue_idx(j, b):
            row0 = base + j * CHUNK
            pltpu.async_copy(b2aq_ref.at[pl.ds(qb + row0, CHUNK)],
                             i1a.at[b], semI[b])
            pltpu.async_copy(b2rq_ref.at[pl.ds(qb + row0, CHUNK)],
                             i2a.at[b], semI[b])
            pltpu.async_copy(tgt_in_ref.at[pl.ds(c * N_BONDS + row0, CHUNK)],
                             tgs[b], semI[b])

        def stage_i(j, b):
            @pl.when(j >= NB)
            def _():
                pltpu.make_async_copy(
                    oba.at[b], acc.at[tgs[b]], semS[b]).wait()
            issue_idx(j, b)

        def stage_g(j, b):
            row0 = base + j * CHUNK
            pltpu.make_async_copy(b2aq_ref.at[pl.ds(qb + row0, CHUNK)],
                                  i1a.at[b], semI[b]).wait()
            pltpu.make_async_copy(b2rq_ref.at[pl.ds(qb + row0, CHUNK)],
                                  i2a.at[b], semI[b]).wait()
            pltpu.make_async_copy(
                tgt_in_ref.at[pl.ds(c * N_BONDS + row0, CHUNK)],
                tgs[b], semI[b]).wait()
            pltpu.async_copy(t1_ref.at[i1a.at[b]], g1a.at[b], semG[b])
            pltpu.async_copy(t2_ref.at[i2a.at[b]], g2a.at[b], semG[b])
            pltpu.async_copy(lin_ref.at[pl.ds(qb + row0, CHUNK)],
                             lba.at[b], semG[b])

        def stage_c(j, b):
            row0 = base + j * CHUNK
            pltpu.make_async_copy(
                t1_ref.at[i1a.at[b]], g1a.at[b], semG[b]).wait()
            pltpu.make_async_copy(
                t2_ref.at[i2a.at[b]], g2a.at[b], semG[b]).wait()
            pltpu.make_async_copy(lin_ref.at[pl.ds(qb + row0, CHUNK)],
                                  lba.at[b], semG[b]).wait()
            if write_h:
                @pl.when(j >= NB)
                def _():
                    pltpu.make_async_copy(
                        oba.at[b], h_ref.at[pl.ds(qb + row0, CHUNK)],
                        semW[b]).wait()

            @pl.loop(0, CHUNK)
            def _(i):
                oba[b, i, :] = jnp.maximum(
                    lba[b, i, :] + g1a[b, i, :] - g2a[b, i, :], 0.0)

            if write_h:
                pltpu.async_copy(
                    oba.at[b], h_ref.at[pl.ds(qb + row0, CHUNK)], semW[b])
            pltpu.async_copy(oba.at[b], acc.at[tgs[b]], semS[b], add=True)

        stage_i(0, 0)
        stage_i(1, 1)
        stage_g(0, 0)

        @pl.loop(0, NCHUNK, step=NB)
        def _(k):
            for bo in range(NB):
                j = k + bo

                @pl.when(j + 1 < NCHUNK)
                def _():
                    stage_g(j + 1, (bo + 1) % NB)

                @pl.when(j + 2 < NCHUNK)
                def _():
                    stage_i(j + 2, (bo + 2) % NB)

                stage_c(j, bo)

        for b in range(NB):
            pltpu.make_async_copy(oba.at[b], acc.at[tgs[b]], semS[b]).wait()
            if write_h:
                pltpu.make_async_copy(
                    oba.at[b], h_ref.at[pl.ds(qb + base, CHUNK)],
                    semW[b]).wait()

        plsc.subcore_barrier()
        _dump_acc(acc, s_ref, q, s)
        if p == 0:
            plsc.subcore_barrier()


def _sc_pass0(b2a, b2a_wide, b2revb, B0q, A0q):
    return pl.kernel(
        _sc_pass0_body,
        out_type=[
            jax.ShapeDtypeStruct((NQ * N_BONDS, QW), _f32),
            jax.ShapeDtypeStruct((NQ * N_ATOMS, QW), _f32),
            jax.ShapeDtypeStruct((NC * N_BONDS,), _i32),
        ],
        mesh=_mesh,
        compiler_params=pltpu.CompilerParams(use_tc_tiling_on_sc=False),
        scratch_types=[
            pltpu.VMEM((CHUNK,), _i32),
            pltpu.VMEM((CHUNK,), _i32),
            pltpu.VMEM((CHUNK,), _i32),
            pltpu.VMEM((CHUNK, 16), _i32),
            pltpu.VMEM((CHUNK, QW), _f32),
            pltpu.VMEM((CHUNK, QW), _f32),
            pltpu.VMEM((CHUNK, QW), _f32),
            pltpu.VMEM((DROWS, QW), _f32),
            pltpu.VMEM_SHARED((N_ATOMS, QW), _f32),
            pltpu.SemaphoreType.DMA,
            pltpu.SemaphoreType.DMA,
            pltpu.SemaphoreType.DMA,
        ],
    )(b2a, b2a_wide, b2revb, B0q, A0q)


def _sc_depth(b2aq, b2rq, tgt, h0q, AM2q, Hhq, write_h):
    out_type = []
    if write_h:
        out_type.append(jax.ShapeDtypeStruct((NQ * N_BONDS, QW), _f32))
    out_type.append(jax.ShapeDtypeStruct((NQ * N_ATOMS, QW), _f32))
    outs = pl.kernel(
        functools.partial(_sc_depth_body, write_h),
        out_type=out_type,
        mesh=_mesh,
        compiler_params=pltpu.CompilerParams(use_tc_tiling_on_sc=False),
        scratch_types=[
            pltpu.VMEM((NB, CHUNK), _i32),
            pltpu.VMEM((NB, CHUNK), _i32),
        ] + [pltpu.VMEM((CHUNK,), _i32) for _ in range(NB)]
        + [
            pltpu.VMEM((NB, CHUNK, QW), _f32),
            pltpu.VMEM((NB, CHUNK, QW), _f32),
            pltpu.VMEM((NB, CHUNK, QW), _f32),
            pltpu.VMEM((NB, CHUNK, QW), _f32),
            pltpu.VMEM((DROWS, QW), _f32),
            pltpu.VMEM_SHARED((N_ATOMS, QW), _f32),
        ] + [pltpu.SemaphoreType.DMA for _ in range(4 * NB)],
    )(b2aq, b2rq, tgt, h0q, AM2q, Hhq)
    if write_h:
        return outs[0], outs[1]
    return None, (outs[0] if isinstance(outs, (tuple, list)) else outs)


def _qspec2(j, nb, blk):
    return pl.BlockSpec((blk, QW), lambda q, i: (j * nb + i, 0))


def _qspec1(j, nb, blk):
    return pl.BlockSpec((blk, QW), lambda i: (j * nb + i, 0))


def _split_w(wt):
    k = wt.shape[0]
    return wt.reshape(k, NQ, QW).transpose(1, 0, 2)


def _mm_split_body(x_ref, w_ref, o_ref):
    o_ref[...] = jnp.dot(x_ref[...], w_ref[0], preferred_element_type=_f32)


def _mm_split(x, wt, blk):
    n, k = x.shape
    nb = n // blk
    return pl.pallas_call(
        _mm_split_body,
        grid=(NQ, nb),
        in_specs=[
            pl.BlockSpec((blk, k), lambda q, i: (i, 0)),
            pl.BlockSpec((1, k, QW), lambda q, i: (q, 0, 0)),
        ],
        out_specs=pl.BlockSpec((blk, QW), lambda q, i: (q * nb + i, 0)),
        out_shape=jax.ShapeDtypeStruct((NQ * n, QW), _f32),
    )(x, _split_w(wt))


def _mm_stacked_body(x0_ref, x1_ref, x2_ref, x3_ref, w_ref, o_ref):
    x = jnp.concatenate(
        [x0_ref[...], x1_ref[...], x2_ref[...], x3_ref[...]], axis=1)
    o_ref[...] = jnp.dot(x, w_ref[0], preferred_element_type=_f32)


def _mm_stacked(xq, wt, n, blk):
    nb = n // blk
    return pl.pallas_call(
        _mm_stacked_body,
        grid=(NQ, nb),
        in_specs=[_qspec2(j, nb, blk) for j in range(NQ)]
        + [pl.BlockSpec((1, HIDDEN, QW), lambda q, i: (q, 0, 0))],
        out_specs=pl.BlockSpec((blk, QW), lambda q, i: (q * nb + i, 0)),
        out_shape=jax.ShapeDtypeStruct((NQ * n, QW), _f32),
    )(xq, xq, xq, xq, _split_w(wt))


def _final_body(x0_ref, x1_ref, x2_ref, x3_ref, fa_ref, wm_ref, wa_ref,
                b_ref, ids_ref, o_ref):
    i = pl.program_id(0)
    x = jnp.concatenate(
        [x0_ref[...], x1_ref[...], x2_ref[...], x3_ref[...]], axis=1)
    y = (jnp.dot(x, wm_ref[...], preferred_element_type=_f32)
         + jnp.dot(fa_ref[...], wa_ref[...], preferred_element_type=_f32)
         + b_ref[...])
    atom_h = jnp.maximum(y, 0.0)
    ids = ids_ref[0, 0, :]
    onehot = (lax.broadcasted_iota(_i32, (NMOLP, atom_h.shape[0]), 0)
              == ids[None, :]).astype(_f32)
    contrib = jnp.dot(onehot, atom_h, preferred_element_type=_f32)

    @pl.when(i == 0)
    def _():
        o_ref[...] = jnp.zeros_like(o_ref)

    o_ref[...] += contrib


def _final_readout(Sq, f_atoms, wm, wa, bias, mol_ids, blk=2000):
    nb = N_ATOMS // blk
    ids3 = mol_ids.reshape(nb, 1, blk)
    return pl.pallas_call(
        _final_body,
        grid=(nb,),
        in_specs=[_qspec1(j, nb, blk) for j in range(NQ)] + [
            pl.BlockSpec((blk, ATOM_FDIM), lambda i: (i, 0)),
            pl.BlockSpec((HIDDEN, HIDDEN), lambda i: (0, 0)),
            pl.BlockSpec((ATOM_FDIM, HIDDEN), lambda i: (0, 0)),
            pl.BlockSpec((1, HIDDEN), lambda i: (0, 0)),
            pl.BlockSpec((1, 1, blk), lambda i: (i, 0, 0)),
        ],
        out_specs=pl.BlockSpec((NMOLP, HIDDEN), lambda i: (0, 0)),
        out_shape=jax.ShapeDtypeStruct((NMOLP, HIDDEN), _f32),
    )(Sq, Sq, Sq, Sq, f_atoms, wm, wa, bias.reshape(1, -1), ids3)


def _head_body(mv_ref, r1w_ref, r1b_ref, r2w_ref, r2b_ref, o_ref):
    out = jnp.maximum(
        jnp.dot(mv_ref[...], r1w_ref[...], preferred_element_type=_f32)
        + r1b_ref[...], 0.0)
    o_ref[...] = (
        jnp.dot(out, r2w_ref[...], preferred_element_type=_f32)
        + r2b_ref[...])


def _head(mol_vecs, R1_t, R1_b, R2_t, R2_b):
    m = mol_vecs.shape[0]
    return pl.pallas_call(
        _head_body,
        out_shape=jax.ShapeDtypeStruct((m, TASKS), _f32),
    )(mol_vecs, R1_t, R1_b.reshape(1, -1), R2_t, R2_b.reshape(1, -1))


def kernel(f_atoms, f_bonds, b2a, b2revb, mol_ids, W_i, W_h, W_o_w, W_o_b,
           R1_w, R1_b, R2_w, R2_b):
    W_ia_t = W_i[:, :ATOM_FDIM].T
    W_ib_t = W_i[:, ATOM_FDIM:].T
    W_h_t = W_h.T
    W_oa_t = W_o_w[:, :ATOM_FDIM].T
    W_om_t = W_o_w[:, ATOM_FDIM:].T

    A0q = _mm_split(f_atoms, W_ia_t, blk=2000)
    B0q = _mm_split(f_bonds, W_ib_t, blk=8000)
    b2a_wide = jnp.broadcast_to(b2a[:, None], (N_BONDS, 16))
    qoff = jnp.arange(NQ, dtype=_i32)
    b2aq = (b2a[None, :] + qoff[:, None] * N_ATOMS).reshape(-1)
    b2rq = (b2revb[None, :] + qoff[:, None] * N_BONDS).reshape(-1)

    h0q, Sq, tgt = _sc_pass0(b2a, b2a_wide, b2revb, B0q, A0q)

    hq = h0q
    for d in range(DEPTH):
        AM2q = _mm_stacked(Sq, W_h_t, N_ATOMS, blk=2000)
        Hhq = _mm_stacked(hq, W_h_t, N_BONDS, blk=8000)
        write_h = d < DEPTH - 1
        if write_h:
            hq, Sq = _sc_depth(b2aq, b2rq, tgt, h0q, AM2q, Hhq, True)
        else:
            _, Sq = _sc_depth(b2aq, b2rq, tgt, h0q, AM2q, Hhq, False)

    mol512 = _final_readout(Sq, f_atoms, W_om_t, W_oa_t, W_o_b, mol_ids)
    return _head(mol512[:N_MOLS], R1_w.T, R1_b, R2_w.T, R2_b)


# speedup vs baseline: 2.4794x; 1.5695x over previous
import functools

import jax
import jax.numpy as jnp
from jax import lax
from jax.experimental import pallas as pl
from jax.experimental.pallas import tpu as pltpu
from jax.experimental.pallas import tpu_sc as plsc

N_ATOMS = 50000
N_BONDS = 800000
ATOM_FDIM = 133
BOND_FDIM = 13
HIDDEN = 64
TASKS = 12
DEPTH = 3
N_MOLS = 500

NC = 2
NS = 16
NQ = 4
QW = HIDDEN // NQ
BPS = N_BONDS // NS
CHUNK = 80
NCHUNK = BPS // CHUNK
DROWS = 3128
DLAST = N_ATOMS - (NS - 1) * DROWS
NMOLP = 512

_mesh = plsc.VectorSubcoreMesh(core_axis_name="c", subcore_axis_name="s",
                               num_cores=2)
_f32 = jnp.float32
_i32 = jnp.int32


def _zero_acc(zb, acc, s):
    @pl.loop(0, DROWS)
    def _(i):
        zb[i, :] = jnp.zeros((QW,), _f32)

    @pl.when(s < NS - 1)
    def _():
        pltpu.sync_copy(zb, acc.at[pl.ds(s * DROWS, DROWS)])

    @pl.when(s == NS - 1)
    def _():
        pltpu.sync_copy(zb.at[pl.ds(0, DLAST)], acc.at[pl.ds(s * DROWS, DLAST)])


def _dump_acc(acc, s_ref, q, s):
    @pl.when(s < NS - 1)
    def _():
        pltpu.sync_copy(acc.at[pl.ds(s * DROWS, DROWS)],
                        s_ref.at[pl.ds(q * N_ATOMS + s * DROWS, DROWS)])

    @pl.when(s == NS - 1)
    def _():
        pltpu.sync_copy(acc.at[pl.ds(s * DROWS, DLAST)],
                        s_ref.at[pl.ds(q * N_ATOMS + s * DROWS, DLAST)])


def _sc_pass0_body(b2aq_ref, b2aw_ref, b2revb_ref, lin_ref, t1_ref,
                   h_ref, s_ref, tgt_ref,
                   i1, i2, tg, tb, g1, lb, ob, zb, acc, sem1, sem2, sem3):
    c = lax.axis_index("c")
    s = lax.axis_index("s")
    base = s * BPS
    lane = jnp.arange(16, dtype=_i32)

    for p in range(2):
        q = 2 * c + p
        _zero_acc(zb, acc, s)
        plsc.subcore_barrier()

        @pl.loop(0, NCHUNK)
        def _(k):
            row0 = base + k * CHUNK
            a1 = pltpu.async_copy(
                b2aq_ref.at[pl.ds(q * N_BONDS + row0, CHUNK)], i1, sem1)

            if p == 0:
                a2 = pltpu.async_copy(
                    b2revb_ref.at[pl.ds(row0, CHUNK)], i2, sem1)
                a1.wait()
                a2.wait()
                tbc = pltpu.async_copy(b2aw_ref.at[i2], tb, sem2)
                tbc.wait()

                @pl.loop(0, CHUNK // 16)
                def _(j):
                    res = jnp.zeros((16,), _i32)
                    for t in range(16):
                        res = jnp.where(lane == t, tb[j * 16 + t, :], res)
                    tg[pl.ds(j * 16, 16)] = res

                tgw = pltpu.async_copy(
                    tg, tgt_ref.at[pl.ds(c * N_BONDS + row0, CHUNK)], sem3)
            else:
                a3 = pltpu.async_copy(
                    tgt_ref.at[pl.ds(c * N_BONDS + row0, CHUNK)], tg, sem1)
                a1.wait()
                a3.wait()

            g1c = pltpu.async_copy(t1_ref.at[i1], g1, sem2)
            lbc = pltpu.async_copy(
                lin_ref.at[pl.ds(q * N_BONDS + row0, CHUNK)], lb, sem2)
            g1c.wait()
            lbc.wait()

            @pl.loop(0, CHUNK)
            def _(i):
                ob[i, :] = jnp.maximum(lb[i, :] + g1[i, :], 0.0)

            hw = pltpu.async_copy(
                ob, h_ref.at[pl.ds(q * N_BONDS + row0, CHUNK)], sem3)
            pltpu.sync_copy(ob, acc.at[tg], add=True)
            hw.wait()
            if p == 0:
                tgw.wait()

        plsc.subcore_barrier()
        _dump_acc(acc, s_ref, q, s)
        if p == 0:
            plsc.subcore_barrier()


NB = 5


def _sc_depth_body(write_h, b2aq_ref, b2rq_ref, tgt_in_ref, lin_ref, t1_ref,
                   t2_ref, *rest):
    if write_h:
        h_ref, s_ref = rest[0], rest[1]
        scratch = rest[2:]
    else:
        s_ref = rest[0]
        scratch = rest[1:]
    i1a, i2a = scratch[0], scratch[1]
    tgs = list(scratch[2:2 + NB])
    g1a, g2a, lba, oba, zb, acc = scratch[2 + NB:8 + NB]
    sems = scratch[8 + NB:]
    semI, semG, semS, semW = (sems[0:NB], sems[NB:2 * NB],
                              sems[2 * NB:3 * NB], sems[3 * NB:4 * NB])
    c = lax.axis_index("c")
    s = lax.axis_index("s")
    base = s * BPS

    for p in range(2):
        q = 2 * c + p
        qb = q * N_BONDS

        _zero_acc(zb, acc, s)
        plsc.subcore_barrier()

        def issue_idx(j, b):
            row0 = base + j * CHUNK
            pltpu.async_copy(b2aq_ref.at[pl.ds(qb + row0, CHUNK)],
                             i1a.at[b], semI[b])
            pltpu.async_copy(b2rq_ref.at[pl.ds(qb + row0, CHUNK)],
                             i2a.at[b], semI[b])
            pltpu.async_copy(tgt_in_ref.at[pl.ds(c * N_BONDS + row0, CHUNK)],
                             tgs[b], semI[b])

        def stage_i(j, b):
            @pl.when(j >= NB)
            def _():
                pltpu.make_async_copy(
                    oba.at[b], acc.at[tgs[b]], semS[b]).wait()
            issue_idx(j, b)

        def stage_g(j, b):
            row0 = base + j * CHUNK
            pltpu.make_async_copy(b2aq_ref.at[pl.ds(qb + row0, CHUNK)],
                                  i1a.at[b], semI[b]).wait()
            pltpu.make_async_copy(b2rq_ref.at[pl.ds(qb + row0, CHUNK)],
                                  i2a.at[b], semI[b]).wait()
            pltpu.make_async_copy(
                tgt_in_ref.at[pl.ds(c * N_BONDS + row0, CHUNK)],
                tgs[b], semI[b]).wait()
            pltpu.async_copy(t1_ref.at[i1a.at[b]], g1a.at[b], semG[b])
            pltpu.async_copy(t2_ref.at[i2a.at[b]], g2a.at[b], semG[b])
            pltpu.async_copy(lin_ref.at[pl.ds(qb + row0, CHUNK)],
                             lba.at[b], semG[b])

        def stage_c(j, b):
            row0 = base + j * CHUNK
            pltpu.make_async_copy(
                t1_ref.at[i1a.at[b]], g1a.at[b], semG[b]).wait()
            pltpu.make_async_copy(
                t2_ref.at[i2a.at[b]], g2a.at[b], semG[b]).wait()
            pltpu.make_async_copy(lin_ref.at[pl.ds(qb + row0, CHUNK)],
                                  lba.at[b], semG[b]).wait()
            if write_h:
                @pl.when(j >= NB)
                def _():
                    pltpu.make_async_copy(
                        oba.at[b], h_ref.at[pl.ds(qb + row0, CHUNK)],
                        semW[b]).wait()

            @pl.loop(0, CHUNK)
            def _(i):
                oba[b, i, :] = jnp.maximum(
                    lba[b, i, :] + g1a[b, i, :] - g2a[b, i, :], 0.0)

            if write_h:
                pltpu.async_copy(
                    oba.at[b], h_ref.at[pl.ds(qb + row0, CHUNK)], semW[b])
            pltpu.async_copy(oba.at[b], acc.at[tgs[b]], semS[b], add=True)

        stage_i(0, 0)
        stage_i(1, 1)
        stage_g(0, 0)

        @pl.loop(0, NCHUNK, step=NB)
        def _(k):
            for bo in range(NB):
                j = k + bo

                @pl.when(j + 1 < NCHUNK)
                def _():
                    stage_g(j + 1, (bo + 1) % NB)

                @pl.when(j + 2 < NCHUNK)
                def _():
                    stage_i(j + 2, (bo + 2) % NB)

                stage_c(j, bo)

        for b in range(NB):
            pltpu.make_async_copy(oba.at[b], acc.at[tgs[b]], semS[b]).wait()
            if write_h:
                pltpu.make_async_copy(
                    oba.at[b], h_ref.at[pl.ds(qb + base, CHUNK)],
                    semW[b]).wait()

        plsc.subcore_barrier()
        _dump_acc(acc, s_ref, q, s)
        if p == 0:
            plsc.subcore_barrier()


def _sc_pass0(b2a, b2a_wide, b2revb, B0q, A0q):
    return pl.kernel(
        _sc_pass0_body,
        out_type=[
            jax.ShapeDtypeStruct((NQ * N_BONDS, QW), _f32),
            jax.ShapeDtypeStruct((NQ * N_ATOMS, QW), _f32),
            jax.ShapeDtypeStruct((NC * N_BONDS,), _i32),
        ],
        mesh=_mesh,
        compiler_params=pltpu.CompilerParams(use_tc_tiling_on_sc=False),
        scratch_types=[
            pltpu.VMEM((CHUNK,), _i32),
            pltpu.VMEM((CHUNK,), _i32),
            pltpu.VMEM((CHUNK,), _i32),
            pltpu.VMEM((CHUNK, 16), _i32),
            pltpu.VMEM((CHUNK, QW), _f32),
            pltpu.VMEM((CHUNK, QW), _f32),
            pltpu.VMEM((CHUNK, QW), _f32),
            pltpu.VMEM((DROWS, QW), _f32),
            pltpu.VMEM_SHARED((N_ATOMS, QW), _f32),
            pltpu.SemaphoreType.DMA,
            pltpu.SemaphoreType.DMA,
            pltpu.SemaphoreType.DMA,
        ],
    )(b2a, b2a_wide, b2revb, B0q, A0q)


def _sc_depth(b2aq, b2rq, tgt, h0q, AM2q, Hhq, write_h):
    out_type = []
    if write_h:
        out_type.append(jax.ShapeDtypeStruct((NQ * N_BONDS, QW), _f32))
    out_type.append(jax.ShapeDtypeStruct((NQ * N_ATOMS, QW), _f32))
    outs = pl.kernel(
        functools.partial(_sc_depth_body, write_h),
        out_type=out_type,
        mesh=_mesh,
        compiler_params=pltpu.CompilerParams(use_tc_tiling_on_sc=False),
        scratch_types=[
            pltpu.VMEM((NB, CHUNK), _i32),
            pltpu.VMEM((NB, CHUNK), _i32),
        ] + [pltpu.VMEM((CHUNK,), _i32) for _ in range(NB)]
        + [
            pltpu.VMEM((NB, CHUNK, QW), _f32),
            pltpu.VMEM((NB, CHUNK, QW), _f32),
            pltpu.VMEM((NB, CHUNK, QW), _f32),
            pltpu.VMEM((NB, CHUNK, QW), _f32),
            pltpu.VMEM((DROWS, QW), _f32),
            pltpu.VMEM_SHARED((N_ATOMS, QW), _f32),
        ] + [pltpu.SemaphoreType.DMA for _ in range(4 * NB)],
    )(b2aq, b2rq, tgt, h0q, AM2q, Hhq)
    if write_h:
        return outs[0], outs[1]
    return None, (outs[0] if isinstance(outs, (tuple, list)) else outs)


def _qspec2(j, nb, blk):
    return pl.BlockSpec((blk, QW), lambda q, i: (j * nb + i, 0))


def _qspec1(j, nb, blk):
    return pl.BlockSpec((blk, QW), lambda i: (j * nb + i, 0))


def _split_w(wt):
    k = wt.shape[0]
    return wt.reshape(k, NQ, QW).transpose(1, 0, 2)


def _mm_split_body(x_ref, w_ref, o_ref):
    o_ref[...] = jnp.dot(x_ref[...], w_ref[0], preferred_element_type=_f32)


def _mm_split(x, wt, blk):
    n, k = x.shape
    nb = n // blk
    return pl.pallas_call(
        _mm_split_body,
        grid=(NQ, nb),
        in_specs=[
            pl.BlockSpec((blk, k), lambda q, i: (i, 0)),
            pl.BlockSpec((1, k, QW), lambda q, i: (q, 0, 0)),
        ],
        out_specs=pl.BlockSpec((blk, QW), lambda q, i: (q * nb + i, 0)),
        out_shape=jax.ShapeDtypeStruct((NQ * n, QW), _f32),
    )(x, _split_w(wt))


def _mm_full_body(x0_ref, x1_ref, x2_ref, x3_ref, w_ref, o_ref):
    x = jnp.concatenate(
        [x0_ref[...], x1_ref[...], x2_ref[...], x3_ref[...]], axis=1)
    o_ref[...] = jnp.dot(x, w_ref[...], preferred_element_type=_f32)


def _mm_full(xq, wt, n, blk):
    nb = n // blk
    return pl.pallas_call(
        _mm_full_body,
        grid=(nb,),
        in_specs=[_qspec1(j, nb, blk) for j in range(NQ)]
        + [pl.BlockSpec((HIDDEN, HIDDEN), lambda i: (0, 0))],
        out_specs=pl.BlockSpec((blk, HIDDEN), lambda i: (i, 0)),
        out_shape=jax.ShapeDtypeStruct((n, HIDDEN), _f32),
    )(xq, xq, xq, xq, wt)


def _mm_body(x_ref, w_ref, o_ref):
    o_ref[...] = jnp.dot(x_ref[...], w_ref[...], preferred_element_type=_f32)


def _mm(x, wt, blk):
    n, k = x.shape
    h = wt.shape[1]
    return pl.pallas_call(
        _mm_body,
        grid=(n // blk,),
        in_specs=[
            pl.BlockSpec((blk, k), lambda i: (i, 0)),
            pl.BlockSpec((k, h), lambda i: (0, 0)),
        ],
        out_specs=pl.BlockSpec((blk, h), lambda i: (i, 0)),
        out_shape=jax.ShapeDtypeStruct((n, h), _f32),
    )(x, wt)


def _final_body(x0_ref, x1_ref, x2_ref, x3_ref, fa_ref, wm_ref, wa_ref,
                b_ref, ids_ref, o_ref):
    i = pl.program_id(0)
    x = jnp.concatenate(
        [x0_ref[...], x1_ref[...], x2_ref[...], x3_ref[...]], axis=1)
    y = (jnp.dot(x, wm_ref[...], preferred_element_type=_f32)
         + jnp.dot(fa_ref[...], wa_ref[...], preferred_element_type=_f32)
         + b_ref[...])
    atom_h = jnp.maximum(y, 0.0)
    ids = ids_ref[0, 0, :]
    onehot = (lax.broadcasted_iota(_i32, (NMOLP, atom_h.shape[0]), 0)
              == ids[None, :]).astype(_f32)
    contrib = jnp.dot(onehot, atom_h, preferred_element_type=_f32)

    @pl.when(i == 0)
    def _():
        o_ref[...] = jnp.zeros_like(o_ref)

    o_ref[...] += contrib


def _final_readout(Sq, f_atoms, wm, wa, bias, mol_ids, blk=2000):
    nb = N_ATOMS // blk
    ids3 = mol_ids.reshape(nb, 1, blk)
    return pl.pallas_call(
        _final_body,
        grid=(nb,),
        in_specs=[_qspec1(j, nb, blk) for j in range(NQ)] + [
            pl.BlockSpec((blk, ATOM_FDIM), lambda i: (i, 0)),
            pl.BlockSpec((HIDDEN, HIDDEN), lambda i: (0, 0)),
            pl.BlockSpec((ATOM_FDIM, HIDDEN), lambda i: (0, 0)),
            pl.BlockSpec((1, HIDDEN), lambda i: (0, 0)),
            pl.BlockSpec((1, 1, blk), lambda i: (i, 0, 0)),
        ],
        out_specs=pl.BlockSpec((NMOLP, HIDDEN), lambda i: (0, 0)),
        out_shape=jax.ShapeDtypeStruct((NMOLP, HIDDEN), _f32),
    )(Sq, Sq, Sq, Sq, f_atoms, wm, wa, bias.reshape(1, -1), ids3)


def _head_body(mv_ref, r1w_ref, r1b_ref, r2w_ref, r2b_ref, o_ref):
    out = jnp.maximum(
        jnp.dot(mv_ref[...], r1w_ref[...], preferred_element_type=_f32)
        + r1b_ref[...], 0.0)
    o_ref[...] = (
        jnp.dot(out, r2w_ref[...], preferred_element_type=_f32)
        + r2b_ref[...])


def _head(mol_vecs, R1_t, R1_b, R2_t, R2_b):
    m = mol_vecs.shape[0]
    return pl.pallas_call(
        _head_body,
        out_shape=jax.ShapeDtypeStruct((m, TASKS), _f32),
    )(mol_vecs, R1_t, R1_b.reshape(1, -1), R2_t, R2_b.reshape(1, -1))


def kernel(f_atoms, f_bonds, b2a, b2revb, mol_ids, W_i, W_h, W_o_w, W_o_b,
           R1_w, R1_b, R2_w, R2_b):
    W_ia_t = W_i[:, :ATOM_FDIM].T
    W_ib_t = W_i[:, ATOM_FDIM:].T
    W_h_t = W_h.T
    W_oa_t = W_o_w[:, :ATOM_FDIM].T
    W_om_t = W_o_w[:, ATOM_FDIM:].T

    A0i = _mm(f_atoms, W_ia_t, blk=2000).reshape(NQ * N_ATOMS, QW)
    B0q = _mm_split(f_bonds, W_ib_t, blk=8000)
    b2a_wide = jnp.broadcast_to(b2a[:, None], (N_BONDS, 16))
    qoff = jnp.arange(NQ, dtype=_i32)
    b2aq = (NQ * b2a[None, :] + qoff[:, None]).reshape(-1)
    b2rq = (NQ * b2revb[None, :] + qoff[:, None]).reshape(-1)

    h0q, Sq, tgt = _sc_pass0(b2aq, b2a_wide, b2revb, B0q, A0i)

    hq = h0q
    for d in range(DEPTH):
        AM2i = _mm_full(Sq, W_h_t, N_ATOMS, blk=2000).reshape(
            NQ * N_ATOMS, QW)
        Hhi = _mm_full(hq, W_h_t, N_BONDS, blk=8000).reshape(
            NQ * N_BONDS, QW)
        write_h = d < DEPTH - 1
        if write_h:
            hq, Sq = _sc_depth(b2aq, b2rq, tgt, h0q, AM2i, Hhi, True)
        else:
            _, Sq = _sc_depth(b2aq, b2rq, tgt, h0q, AM2i, Hhi, False)

    mol512 = _final_readout(Sq, f_atoms, W_om_t, W_oa_t, W_o_b, mol_ids)
    return _head(mol512[:N_MOLS], R1_w.T, R1_b, R2_w.T, R2_b)


# speedup vs baseline: 2.8167x; 1.1360x over previous
import functools

import jax
import jax.numpy as jnp
from jax import lax
from jax.experimental import pallas as pl
from jax.experimental.pallas import tpu as pltpu
from jax.experimental.pallas import tpu_sc as plsc

N_ATOMS = 50000
N_BONDS = 800000
ATOM_FDIM = 133
BOND_FDIM = 13
HIDDEN = 64
TASKS = 12
DEPTH = 3
N_MOLS = 500

NC = 2
NS = 16
NQ = 4
QW = HIDDEN // NQ
BPS = N_BONDS // NS
CHUNK = 80
NCHUNK = BPS // CHUNK
DROWS = 3128
DLAST = N_ATOMS - (NS - 1) * DROWS
NMOLP = 512

_mesh = plsc.VectorSubcoreMesh(core_axis_name="c", subcore_axis_name="s",
                               num_cores=2)
_f32 = jnp.float32
_i32 = jnp.int32


def _zero_acc(zb, acc, s):
    @pl.loop(0, DROWS)
    def _(i):
        zb[i, :] = jnp.zeros((QW,), _f32)

    @pl.when(s < NS - 1)
    def _():
        pltpu.sync_copy(zb, acc.at[pl.ds(s * DROWS, DROWS)])

    @pl.when(s == NS - 1)
    def _():
        pltpu.sync_copy(zb.at[pl.ds(0, DLAST)], acc.at[pl.ds(s * DROWS, DLAST)])


def _dump_acc(acc, s_ref, q, s):
    @pl.when(s < NS - 1)
    def _():
        pltpu.sync_copy(acc.at[pl.ds(s * DROWS, DROWS)],
                        s_ref.at[pl.ds(q * N_ATOMS + s * DROWS, DROWS)])

    @pl.when(s == NS - 1)
    def _():
        pltpu.sync_copy(acc.at[pl.ds(s * DROWS, DLAST)],
                        s_ref.at[pl.ds(q * N_ATOMS + s * DROWS, DLAST)])


def _sc_pass0_body(b2aq_ref, b2aw_ref, b2revb_ref, lin_ref, t1_ref,
                   h_ref, s_ref, tgt_ref, *scratch):
    i1a, i2a = scratch[0], scratch[1]
    tgs = list(scratch[2:2 + NB])
    tba, g1a, lba, oba, zb, acc = scratch[2 + NB:8 + NB]
    sems = scratch[8 + NB:]
    semI, semG, semS, semW, semT = (
        sems[0:NB], sems[NB:2 * NB], sems[2 * NB:3 * NB],
        sems[3 * NB:4 * NB], sems[4 * NB:5 * NB])
    c = lax.axis_index("c")
    s = lax.axis_index("s")
    base = s * BPS
    lane = jnp.arange(16, dtype=_i32)

    for p in range(2):
        q = 2 * c + p
        qb = q * N_BONDS

        _zero_acc(zb, acc, s)
        plsc.subcore_barrier()

        def stage_i(j, b):
            row0 = base + j * CHUNK

            @pl.when(j >= NB)
            def _():
                pltpu.make_async_copy(
                    oba.at[b], acc.at[tgs[b]], semS[b]).wait()
            pltpu.async_copy(b2aq_ref.at[pl.ds(qb + row0, CHUNK)],
                             i1a.at[b], semI[b])
            if p == 0:
                pltpu.async_copy(b2revb_ref.at[pl.ds(row0, CHUNK)],
                                 i2a.at[b], semI[b])
            else:
                pltpu.async_copy(
                    tgt_ref.at[pl.ds(c * N_BONDS + row0, CHUNK)],
                    tgs[b], semI[b])

        def stage_g(j, b):
            row0 = base + j * CHUNK
            pltpu.make_async_copy(b2aq_ref.at[pl.ds(qb + row0, CHUNK)],
                                  i1a.at[b], semI[b]).wait()
            if p == 0:
                pltpu.make_async_copy(b2revb_ref.at[pl.ds(row0, CHUNK)],
                                      i2a.at[b], semI[b]).wait()
                pltpu.async_copy(b2aw_ref.at[i2a.at[b]], tba.at[b], semG[b])
            else:
                pltpu.make_async_copy(
                    tgt_ref.at[pl.ds(c * N_BONDS + row0, CHUNK)],
                    tgs[b], semI[b]).wait()
            pltpu.async_copy(t1_ref.at[i1a.at[b]], g1a.at[b], semG[b])
            pltpu.async_copy(lin_ref.at[pl.ds(qb + row0, CHUNK)],
                             lba.at[b], semG[b])

        def stage_c(j, b):
            row0 = base + j * CHUNK
            pltpu.make_async_copy(
                t1_ref.at[i1a.at[b]], g1a.at[b], semG[b]).wait()
            pltpu.make_async_copy(lin_ref.at[pl.ds(qb + row0, CHUNK)],
                                  lba.at[b], semG[b]).wait()
            if p == 0:
                pltpu.make_async_copy(
                    b2aw_ref.at[i2a.at[b]], tba.at[b], semG[b]).wait()

                @pl.when(j >= NB)
                def _():
                    pltpu.make_async_copy(
                        tgs[b],
                        tgt_ref.at[pl.ds(c * N_BONDS + row0, CHUNK)],
                        semT[b]).wait()

                @pl.loop(0, CHUNK // 16)
                def _(jj):
                    res = jnp.zeros((16,), _i32)
                    for t in range(16):
                        res = jnp.where(lane == t, tba[b, jj * 16 + t, :],
                                        res)
                    tgs[b][pl.ds(jj * 16, 16)] = res

                pltpu.async_copy(
                    tgs[b], tgt_ref.at[pl.ds(c * N_BONDS + row0, CHUNK)],
                    semT[b])

            @pl.when(j >= NB)
            def _():
                pltpu.make_async_copy(
                    oba.at[b], h_ref.at[pl.ds(qb + row0, CHUNK)],
                    semW[b]).wait()

            @pl.loop(0, CHUNK)
            def _(i):
                oba[b, i, :] = jnp.maximum(lba[b, i, :] + g1a[b, i, :], 0.0)

            pltpu.async_copy(
                oba.at[b], h_ref.at[pl.ds(qb + row0, CHUNK)], semW[b])
            pltpu.async_copy(oba.at[b], acc.at[tgs[b]], semS[b], add=True)

        stage_i(0, 0)
        stage_i(1, 1)
        stage_g(0, 0)

        @pl.loop(0, NCHUNK, step=NB)
        def _(k):
            for bo in range(NB):
                j = k + bo

                @pl.when(j + 1 < NCHUNK)
                def _():
                    stage_g(j + 1, (bo + 1) % NB)

                @pl.when(j + 2 < NCHUNK)
                def _():
                    stage_i(j + 2, (bo + 2) % NB)

                stage_c(j, bo)

        for b in range(NB):
            pltpu.make_async_copy(oba.at[b], acc.at[tgs[b]], semS[b]).wait()
            pltpu.make_async_copy(
                oba.at[b], h_ref.at[pl.ds(qb + base, CHUNK)], semW[b]).wait()
            if p == 0:
                pltpu.make_async_copy(
                    tgs[b], tgt_ref.at[pl.ds(c * N_BONDS + base, CHUNK)],
                    semT[b]).wait()

        plsc.subcore_barrier()
        _dump_acc(acc, s_ref, q, s)
        if p == 0:
            plsc.subcore_barrier()


NB = 5


def _sc_depth_body(write_h, b2aq_ref, b2rq_ref, tgt_in_ref, lin_ref, t1_ref,
                   t2_ref, *rest):
    if write_h:
        h_ref, s_ref = rest[0], rest[1]
        scratch = rest[2:]
    else:
        s_ref = rest[0]
        scratch = rest[1:]
    i1a, i2a = scratch[0], scratch[1]
    tgs = list(scratch[2:2 + NB])
    g1a, g2a, lba, oba, zb, acc = scratch[2 + NB:8 + NB]
    sems = scratch[8 + NB:]
    semI, semG, semS, semW = (sems[0:NB], sems[NB:2 * NB],
                              sems[2 * NB:3 * NB], sems[3 * NB:4 * NB])
    c = lax.axis_index("c")
    s = lax.axis_index("s")
    base = s * BPS

    for p in range(2):
        q = 2 * c + p
        qb = q * N_BONDS

        _zero_acc(zb, acc, s)
        plsc.subcore_barrier()

        def issue_idx(j, b):
            row0 = base + j * CHUNK
            pltpu.async_copy(b2aq_ref.at[pl.ds(qb + row0, CHUNK)],
                             i1a.at[b], semI[b])
            pltpu.async_copy(b2rq_ref.at[pl.ds(qb + row0, CHUNK)],
                             i2a.at[b], semI[b])
            pltpu.async_copy(tgt_in_ref.at[pl.ds(c * N_BONDS + row0, CHUNK)],
                             tgs[b], semI[b])

        def stage_i(j, b):
            @pl.when(j >= NB)
            def _():
                pltpu.make_async_copy(
                    oba.at[b], acc.at[tgs[b]], semS[b]).wait()
            issue_idx(j, b)

        def stage_g(j, b):
            row0 = base + j * CHUNK
            pltpu.make_async_copy(b2aq_ref.at[pl.ds(qb + row0, CHUNK)],
                                  i1a.at[b], semI[b]).wait()
            pltpu.make_async_copy(b2rq_ref.at[pl.ds(qb + row0, CHUNK)],
                                  i2a.at[b], semI[b]).wait()
            pltpu.make_async_copy(
                tgt_in_ref.at[pl.ds(c * N_BONDS + row0, CHUNK)],
                tgs[b], semI[b]).wait()
            pltpu.async_copy(t1_ref.at[i1a.at[b]], g1a.at[b], semG[b])
            pltpu.async_copy(t2_ref.at[i2a.at[b]], g2a.at[b], semG[b])
            pltpu.async_copy(lin_ref.at[pl.ds(qb + row0, CHUNK)],
                             lba.at[b], semG[b])

        def stage_c(j, b):
            row0 = base + j * CHUNK
            pltpu.make_async_copy(
                t1_ref.at[i1a.at[b]], g1a.at[b], semG[b]).wait()
            pltpu.make_async_copy(
                t2_ref.at[i2a.at[b]], g2a.at[b], semG[b]).wait()
            pltpu.make_async_copy(lin_ref.at[pl.ds(qb + row0, CHUNK)],
                                  lba.at[b], semG[b]).wait()
            if write_h:
                @pl.when(j >= NB)
                def _():
                    pltpu.make_async_copy(
                        oba.at[b], h_ref.at[pl.ds(qb + row0, CHUNK)],
                        semW[b]).wait()

            @pl.loop(0, CHUNK)
            def _(i):
                oba[b, i, :] = jnp.maximum(
                    lba[b, i, :] + g1a[b, i, :] - g2a[b, i, :], 0.0)

            if write_h:
                pltpu.async_copy(
                    oba.at[b], h_ref.at[pl.ds(qb + row0, CHUNK)], semW[b])
            pltpu.async_copy(oba.at[b], acc.at[tgs[b]], semS[b], add=True)

        stage_i(0, 0)
        stage_i(1, 1)
        stage_g(0, 0)

        @pl.loop(0, NCHUNK, step=NB)
        def _(k):
            for bo in range(NB):
                j = k + bo

                @pl.when(j + 1 < NCHUNK)
                def _():
                    stage_g(j + 1, (bo + 1) % NB)

                @pl.when(j + 2 < NCHUNK)
                def _():
                    stage_i(j + 2, (bo + 2) % NB)

                stage_c(j, bo)

        for b in range(NB):
            pltpu.make_async_copy(oba.at[b], acc.at[tgs[b]], semS[b]).wait()
            if write_h:
                pltpu.make_async_copy(
                    oba.at[b], h_ref.at[pl.ds(qb + base, CHUNK)],
                    semW[b]).wait()

        plsc.subcore_barrier()
        _dump_acc(acc, s_ref, q, s)
        if p == 0:
            plsc.subcore_barrier()


def _sc_pass0(b2aq, b2a_wide, b2revb, B0q, A0i):
    return pl.kernel(
        _sc_pass0_body,
        out_type=[
            jax.ShapeDtypeStruct((NQ * N_BONDS, QW), _f32),
            jax.ShapeDtypeStruct((NQ * N_ATOMS, QW), _f32),
            jax.ShapeDtypeStruct((NC * N_BONDS,), _i32),
        ],
        mesh=_mesh,
        compiler_params=pltpu.CompilerParams(use_tc_tiling_on_sc=False),
        scratch_types=[
            pltpu.VMEM((NB, CHUNK), _i32),
            pltpu.VMEM((NB, CHUNK), _i32),
        ] + [pltpu.VMEM((CHUNK,), _i32) for _ in range(NB)]
        + [
            pltpu.VMEM((NB, CHUNK, 16), _i32),
            pltpu.VMEM((NB, CHUNK, QW), _f32),
            pltpu.VMEM((NB, CHUNK, QW), _f32),
            pltpu.VMEM((NB, CHUNK, QW), _f32),
            pltpu.VMEM((DROWS, QW), _f32),
            pltpu.VMEM_SHARED((N_ATOMS, QW), _f32),
        ] + [pltpu.SemaphoreType.DMA for _ in range(5 * NB)],
    )(b2aq, b2a_wide, b2revb, B0q, A0i)


def _sc_depth(b2aq, b2rq, tgt, h0q, AM2q, Hhq, write_h):
    out_type = []
    if write_h:
        out_type.append(jax.ShapeDtypeStruct((NQ * N_BONDS, QW), _f32))
    out_type.append(jax.ShapeDtypeStruct((NQ * N_ATOMS, QW), _f32))
    outs = pl.kernel(
        functools.partial(_sc_depth_body, write_h),
        out_type=out_type,
        mesh=_mesh,
        compiler_params=pltpu.CompilerParams(use_tc_tiling_on_sc=False),
        scratch_types=[
            pltpu.VMEM((NB, CHUNK), _i32),
            pltpu.VMEM((NB, CHUNK), _i32),
        ] + [pltpu.VMEM((CHUNK,), _i32) for _ in range(NB)]
        + [
            pltpu.VMEM((NB, CHUNK, QW), _f32),
            pltpu.VMEM((NB, CHUNK, QW), _f32),
            pltpu.VMEM((NB, CHUNK, QW), _f32),
            pltpu.VMEM((NB, CHUNK, QW), _f32),
            pltpu.VMEM((DROWS, QW), _f32),
            pltpu.VMEM_SHARED((N_ATOMS, QW), _f32),
        ] + [pltpu.SemaphoreType.DMA for _ in range(4 * NB)],
    )(b2aq, b2rq, tgt, h0q, AM2q, Hhq)
    if write_h:
        return outs[0], outs[1]
    return None, (outs[0] if isinstance(outs, (tuple, list)) else outs)


def _qspec2(j, nb, blk):
    return pl.BlockSpec((blk, QW), lambda q, i: (j * nb + i, 0))


def _qspec1(j, nb, blk):
    return pl.BlockSpec((blk, QW), lambda i: (j * nb + i, 0))


def _split_w(wt):
    k = wt.shape[0]
    return wt.reshape(k, NQ, QW).transpose(1, 0, 2)


def _mm_split_body(x_ref, w_ref, o_ref):
    o_ref[...] = jnp.dot(x_ref[...], w_ref[0], preferred_element_type=_f32)


def _mm_split(x, wt, blk):
    n, k = x.shape
    nb = n // blk
    return pl.pallas_call(
        _mm_split_body,
        grid=(NQ, nb),
        in_specs=[
            pl.BlockSpec((blk, k), lambda q, i: (i, 0)),
            pl.BlockSpec((1, k, QW), lambda q, i: (q, 0, 0)),
        ],
        out_specs=pl.BlockSpec((blk, QW), lambda q, i: (q * nb + i, 0)),
        out_shape=jax.ShapeDtypeStruct((NQ * n, QW), _f32),
    )(x, _split_w(wt))


def _mm_full_body(x0_ref, x1_ref, x2_ref, x3_ref, w_ref, o_ref):
    x = jnp.concatenate(
        [x0_ref[...], x1_ref[...], x2_ref[...], x3_ref[...]], axis=1)
    o_ref[...] = jnp.dot(x, w_ref[...], preferred_element_type=_f32)


def _mm_full(xq, wt, n, blk):
    nb = n // blk
    return pl.pallas_call(
        _mm_full_body,
        grid=(nb,),
        in_specs=[_qspec1(j, nb, blk) for j in range(NQ)]
        + [pl.BlockSpec((HIDDEN, HIDDEN), lambda i: (0, 0))],
        out_specs=pl.BlockSpec((blk, HIDDEN), lambda i: (i, 0)),
        out_shape=jax.ShapeDtypeStruct((n, HIDDEN), _f32),
    )(xq, xq, xq, xq, wt)


def _mm_body(x_ref, w_ref, o_ref):
    o_ref[...] = jnp.dot(x_ref[...], w_ref[...], preferred_element_type=_f32)


def _mm(x, wt, blk):
    n, k = x.shape
    h = wt.shape[1]
    return pl.pallas_call(
        _mm_body,
        grid=(n // blk,),
        in_specs=[
            pl.BlockSpec((blk, k), lambda i: (i, 0)),
            pl.BlockSpec((k, h), lambda i: (0, 0)),
        ],
        out_specs=pl.BlockSpec((blk, h), lambda i: (i, 0)),
        out_shape=jax.ShapeDtypeStruct((n, h), _f32),
    )(x, wt)


def _final_body(x0_ref, x1_ref, x2_ref, x3_ref, fa_ref, wm_ref, wa_ref,
                b_ref, ids_ref, o_ref):
    i = pl.program_id(0)
    x = jnp.concatenate(
        [x0_ref[...], x1_ref[...], x2_ref[...], x3_ref[...]], axis=1)
    y = (jnp.dot(x, wm_ref[...], preferred_element_type=_f32)
         + jnp.dot(fa_ref[...], wa_ref[...], preferred_element_type=_f32)
         + b_ref[...])
    atom_h = jnp.maximum(y, 0.0)
    ids = ids_ref[0, 0, :]
    onehot = (lax.broadcasted_iota(_i32, (NMOLP, atom_h.shape[0]), 0)
              == ids[None, :]).astype(_f32)
    contrib = jnp.dot(onehot, atom_h, preferred_element_type=_f32)

    @pl.when(i == 0)
    def _():
        o_ref[...] = jnp.zeros_like(o_ref)

    o_ref[...] += contrib


def _final_readout(Sq, f_atoms, wm, wa, bias, mol_ids, blk=2000):
    nb = N_ATOMS // blk
    ids3 = mol_ids.reshape(nb, 1, blk)
    return pl.pallas_call(
        _final_body,
        grid=(nb,),
        in_specs=[_qspec1(j, nb, blk) for j in range(NQ)] + [
            pl.BlockSpec((blk, ATOM_FDIM), lambda i: (i, 0)),
            pl.BlockSpec((HIDDEN, HIDDEN), lambda i: (0, 0)),
            pl.BlockSpec((ATOM_FDIM, HIDDEN), lambda i: (0, 0)),
            pl.BlockSpec((1, HIDDEN), lambda i: (0, 0)),
            pl.BlockSpec((1, 1, blk), lambda i: (i, 0, 0)),
        ],
        out_specs=pl.BlockSpec((NMOLP, HIDDEN), lambda i: (0, 0)),
        out_shape=jax.ShapeDtypeStruct((NMOLP, HIDDEN), _f32),
    )(Sq, Sq, Sq, Sq, f_atoms, wm, wa, bias.reshape(1, -1), ids3)


def _head_body(mv_ref, r1w_ref, r1b_ref, r2w_ref, r2b_ref, o_ref):
    out = jnp.maximum(
        jnp.dot(mv_ref[...], r1w_ref[...], preferred_element_type=_f32)
        + r1b_ref[...], 0.0)
    o_ref[...] = (
        jnp.dot(out, r2w_ref[...], preferred_element_type=_f32)
        + r2b_ref[...])


def _head(mol_vecs, R1_t, R1_b, R2_t, R2_b):
    m = mol_vecs.shape[0]
    return pl.pallas_call(
        _head_body,
        out_shape=jax.ShapeDtypeStruct((m, TASKS), _f32),
    )(mol_vecs, R1_t, R1_b.reshape(1, -1), R2_t, R2_b.reshape(1, -1))


def kernel(f_atoms, f_bonds, b2a, b2revb, mol_ids, W_i, W_h, W_o_w, W_o_b,
           R1_w, R1_b, R2_w, R2_b):
    W_ia_t = W_i[:, :ATOM_FDIM].T
    W_ib_t = W_i[:, ATOM_FDIM:].T
    W_h_t = W_h.T
    W_oa_t = W_o_w[:, :ATOM_FDIM].T
    W_om_t = W_o_w[:, ATOM_FDIM:].T

    A0i = _mm(f_atoms, W_ia_t, blk=2000).reshape(NQ * N_ATOMS, QW)
    B0q = _mm_split(f_bonds, W_ib_t, blk=8000)
    b2a_wide = jnp.broadcast_to(b2a[:, None], (N_BONDS, 16))
    qoff = jnp.arange(NQ, dtype=_i32)
    b2aq = (NQ * b2a[None, :] + qoff[:, None]).reshape(-1)
    b2rq = (NQ * b2revb[None, :] + qoff[:, None]).reshape(-1)

    h0q, Sq, tgt = _sc_pass0(b2aq, b2a_wide, b2revb, B0q, A0i)

    hq = h0q
    for d in range(DEPTH):
        AM2i = _mm_full(Sq, W_h_t, N_ATOMS, blk=2000).reshape(
            NQ * N_ATOMS, QW)
        Hhi = _mm_full(hq, W_h_t, N_BONDS, blk=8000).reshape(
            NQ * N_BONDS, QW)
        write_h = d < DEPTH - 1
        if write_h:
            hq, Sq = _sc_depth(b2aq, b2rq, tgt, h0q, AM2i, Hhi, True)
        else:
            _, Sq = _sc_depth(b2aq, b2rq, tgt, h0q, AM2i, Hhi, False)

    mol512 = _final_readout(Sq, f_atoms, W_om_t, W_oa_t, W_o_b, mol_ids)
    return _head(mol512[:N_MOLS], R1_w.T, R1_b, R2_w.T, R2_b)


# speedup vs baseline: 2.8174x; 1.0002x over previous
import functools

import jax
import jax.numpy as jnp
from jax import lax
from jax.experimental import pallas as pl
from jax.experimental.pallas import tpu as pltpu
from jax.experimental.pallas import tpu_sc as plsc

N_ATOMS = 50000
N_BONDS = 800000
ATOM_FDIM = 133
BOND_FDIM = 13
HIDDEN = 64
TASKS = 12
DEPTH = 3
N_MOLS = 500

NC = 2
NS = 16
NQ = 4
QW = HIDDEN // NQ
BPS = N_BONDS // NS
CHUNK = 80
NCHUNK = BPS // CHUNK
DROWS = 3128
DLAST = N_ATOMS - (NS - 1) * DROWS
NMOLP = 512

_mesh = plsc.VectorSubcoreMesh(core_axis_name="c", subcore_axis_name="s",
                               num_cores=2)
_f32 = jnp.float32
_i32 = jnp.int32


def _zero_acc(zb, acc, s):
    @pl.loop(0, DROWS)
    def _(i):
        zb[i, :] = jnp.zeros((QW,), _f32)

    @pl.when(s < NS - 1)
    def _():
        pltpu.sync_copy(zb, acc.at[pl.ds(s * DROWS, DROWS)])

    @pl.when(s == NS - 1)
    def _():
        pltpu.sync_copy(zb.at[pl.ds(0, DLAST)], acc.at[pl.ds(s * DROWS, DLAST)])


def _dump_acc(acc, s_ref, q, s):
    @pl.when(s < NS - 1)
    def _():
        pltpu.sync_copy(acc.at[pl.ds(s * DROWS, DROWS)],
                        s_ref.at[pl.ds(q * N_ATOMS + s * DROWS, DROWS)])

    @pl.when(s == NS - 1)
    def _():
        pltpu.sync_copy(acc.at[pl.ds(s * DROWS, DLAST)],
                        s_ref.at[pl.ds(q * N_ATOMS + s * DROWS, DLAST)])


def _sc_pass0_body(b2aq_ref, b2aw_ref, b2revb_ref, lin_ref, t1_ref,
                   h_ref, s_ref, tgt_ref, *scratch):
    i1a, i2a = scratch[0], scratch[1]
    tgs = list(scratch[2:2 + NB])
    tba, g1a, lba, oba, zb, acc = scratch[2 + NB:8 + NB]
    sems = scratch[8 + NB:]
    semI, semG, semS, semW, semT = (
        sems[0:NB], sems[NB:2 * NB], sems[2 * NB:3 * NB],
        sems[3 * NB:4 * NB], sems[4 * NB:5 * NB])
    c = lax.axis_index("c")
    s = lax.axis_index("s")
    base = s * BPS
    lane = jnp.arange(16, dtype=_i32)

    for p in range(2):
        q = 2 * c + p
        qb = q * N_BONDS

        _zero_acc(zb, acc, s)
        plsc.subcore_barrier()

        def stage_i(j, b):
            row0 = base + j * CHUNK

            @pl.when(j >= NB)
            def _():
                pltpu.make_async_copy(
                    oba.at[b], acc.at[tgs[b]], semS[b]).wait()
            pltpu.async_copy(b2aq_ref.at[pl.ds(qb + row0, CHUNK)],
                             i1a.at[b], semI[b])
            if p == 0:
                pltpu.async_copy(b2revb_ref.at[pl.ds(row0, CHUNK)],
                                 i2a.at[b], semI[b])
            else:
                pltpu.async_copy(
                    tgt_ref.at[pl.ds(c * N_BONDS + row0, CHUNK)],
                    tgs[b], semI[b])

        def stage_g(j, b):
            row0 = base + j * CHUNK
            pltpu.make_async_copy(b2aq_ref.at[pl.ds(qb + row0, CHUNK)],
                                  i1a.at[b], semI[b]).wait()
            if p == 0:
                pltpu.make_async_copy(b2revb_ref.at[pl.ds(row0, CHUNK)],
                                      i2a.at[b], semI[b]).wait()
                pltpu.async_copy(b2aw_ref.at[i2a.at[b]], tba.at[b], semG[b])
            else:
                pltpu.make_async_copy(
                    tgt_ref.at[pl.ds(c * N_BONDS + row0, CHUNK)],
                    tgs[b], semI[b]).wait()
            pltpu.async_copy(t1_ref.at[i1a.at[b]], g1a.at[b], semG[b])
            pltpu.async_copy(lin_ref.at[pl.ds(qb + row0, CHUNK)],
                             lba.at[b], semG[b])

        def stage_c(j, b):
            row0 = base + j * CHUNK
            pltpu.make_async_copy(
                t1_ref.at[i1a.at[b]], g1a.at[b], semG[b]).wait()
            pltpu.make_async_copy(lin_ref.at[pl.ds(qb + row0, CHUNK)],
                                  lba.at[b], semG[b]).wait()
            if p == 0:
                pltpu.make_async_copy(
                    b2aw_ref.at[i2a.at[b]], tba.at[b], semG[b]).wait()

                @pl.when(j >= NB)
                def _():
                    pltpu.make_async_copy(
                        tgs[b],
                        tgt_ref.at[pl.ds(c * N_BONDS + row0, CHUNK)],
                        semT[b]).wait()

                @pl.loop(0, CHUNK // 16)
                def _(jj):
                    res = jnp.zeros((16,), _i32)
                    for t in range(16):
                        res = jnp.where(lane == t, tba[b, jj * 16 + t, :],
                                        res)
                    tgs[b][pl.ds(jj * 16, 16)] = res

                pltpu.async_copy(
                    tgs[b], tgt_ref.at[pl.ds(c * N_BONDS + row0, CHUNK)],
                    semT[b])

            @pl.when(j >= NB)
            def _():
                pltpu.make_async_copy(
                    oba.at[b], h_ref.at[pl.ds(qb + row0, CHUNK)],
                    semW[b]).wait()

            @pl.loop(0, CHUNK)
            def _(i):
                oba[b, i, :] = jnp.maximum(lba[b, i, :] + g1a[b, i, :], 0.0)

            pltpu.async_copy(
                oba.at[b], h_ref.at[pl.ds(qb + row0, CHUNK)], semW[b])
            pltpu.async_copy(oba.at[b], acc.at[tgs[b]], semS[b], add=True)

        stage_i(0, 0)
        stage_i(1, 1)
        stage_g(0, 0)

        @pl.loop(0, NCHUNK, step=NB)
        def _(k):
            for bo in range(NB):
                j = k + bo

                @pl.when(j + 1 < NCHUNK)
                def _():
                    stage_g(j + 1, (bo + 1) % NB)

                @pl.when(j + 2 < NCHUNK)
                def _():
                    stage_i(j + 2, (bo + 2) % NB)

                stage_c(j, bo)

        for b in range(NB):
            pltpu.make_async_copy(oba.at[b], acc.at[tgs[b]], semS[b]).wait()
            pltpu.make_async_copy(
                oba.at[b], h_ref.at[pl.ds(qb + base, CHUNK)], semW[b]).wait()
            if p == 0:
                pltpu.make_async_copy(
                    tgs[b], tgt_ref.at[pl.ds(c * N_BONDS + base, CHUNK)],
                    semT[b]).wait()

        plsc.subcore_barrier()
        _dump_acc(acc, s_ref, q, s)
        if p == 0:
            plsc.subcore_barrier()


NB = 5


def _sc_depth_body(write_h, b2aq_ref, b2rq_ref, tgt_in_ref, lin_ref, t1_ref,
                   t2_ref, *rest):
    if write_h:
        h_ref, s_ref = rest[0], rest[1]
        scratch = rest[2:]
    else:
        s_ref = rest[0]
        scratch = rest[1:]
    i1a, i2a = scratch[0], scratch[1]
    tgs = list(scratch[2:2 + NB])
    g1a, g2a, lba, oba, zb, acc = scratch[2 + NB:8 + NB]
    sems = scratch[8 + NB:]
    semI, semG, semS, semW = (sems[0:NB], sems[NB:2 * NB],
                              sems[2 * NB:3 * NB], sems[3 * NB:4 * NB])
    c = lax.axis_index("c")
    s = lax.axis_index("s")
    base = s * BPS

    for p in range(2):
        q = 2 * c + p
        qb = q * N_BONDS

        _zero_acc(zb, acc, s)
        plsc.subcore_barrier()

        def issue_idx(j, b):
            row0 = base + j * CHUNK
            pltpu.async_copy(b2aq_ref.at[pl.ds(qb + row0, CHUNK)],
                             i1a.at[b], semI[b])
            pltpu.async_copy(b2rq_ref.at[pl.ds(qb + row0, CHUNK)],
                             i2a.at[b], semI[b])
            pltpu.async_copy(tgt_in_ref.at[pl.ds(c * N_BONDS + row0, CHUNK)],
                             tgs[b], semI[b])

        def stage_i(j, b):
            @pl.when(j >= NB)
            def _():
                pltpu.make_async_copy(
                    oba.at[b], acc.at[tgs[b]], semS[b]).wait()
            issue_idx(j, b)

        def stage_g(j, b):
            row0 = base + j * CHUNK
            pltpu.make_async_copy(b2aq_ref.at[pl.ds(qb + row0, CHUNK)],
                                  i1a.at[b], semI[b]).wait()
            pltpu.make_async_copy(b2rq_ref.at[pl.ds(qb + row0, CHUNK)],
                                  i2a.at[b], semI[b]).wait()
            pltpu.make_async_copy(
                tgt_in_ref.at[pl.ds(c * N_BONDS + row0, CHUNK)],
                tgs[b], semI[b]).wait()
            pltpu.async_copy(t1_ref.at[i1a.at[b]], g1a.at[b], semG[b])
            pltpu.async_copy(t2_ref.at[i2a.at[b]], g2a.at[b], semG[b])
            pltpu.async_copy(lin_ref.at[pl.ds(qb + row0, CHUNK)],
                             lba.at[b], semG[b])

        def stage_c(j, b):
            row0 = base + j * CHUNK
            pltpu.make_async_copy(
                t1_ref.at[i1a.at[b]], g1a.at[b], semG[b]).wait()
            pltpu.make_async_copy(
                t2_ref.at[i2a.at[b]], g2a.at[b], semG[b]).wait()
            pltpu.make_async_copy(lin_ref.at[pl.ds(qb + row0, CHUNK)],
                                  lba.at[b], semG[b]).wait()
            if write_h:
                @pl.when(j >= NB)
                def _():
                    pltpu.make_async_copy(
                        oba.at[b], h_ref.at[pl.ds(qb + row0, CHUNK)],
                        semW[b]).wait()

            @pl.loop(0, CHUNK)
            def _(i):
                oba[b, i, :] = jnp.maximum(
                    lba[b, i, :] + g1a[b, i, :] - g2a[b, i, :], 0.0)

            if write_h:
                pltpu.async_copy(
                    oba.at[b], h_ref.at[pl.ds(qb + row0, CHUNK)], semW[b])
            pltpu.async_copy(oba.at[b], acc.at[tgs[b]], semS[b], add=True)

        stage_i(0, 0)
        stage_i(1, 1)
        stage_g(0, 0)

        @pl.loop(0, NCHUNK, step=NB)
        def _(k):
            for bo in range(NB):
                j = k + bo

                @pl.when(j + 1 < NCHUNK)
                def _():
                    stage_g(j + 1, (bo + 1) % NB)

                @pl.when(j + 2 < NCHUNK)
                def _():
                    stage_i(j + 2, (bo + 2) % NB)

                stage_c(j, bo)

        for b in range(NB):
            pltpu.make_async_copy(oba.at[b], acc.at[tgs[b]], semS[b]).wait()
            if write_h:
                pltpu.make_async_copy(
                    oba.at[b], h_ref.at[pl.ds(qb + base, CHUNK)],
                    semW[b]).wait()

        plsc.subcore_barrier()
        _dump_acc(acc, s_ref, q, s)
        if p == 0:
            plsc.subcore_barrier()


def _sc_pass0(b2aq, b2a_wide, b2revb, B0q, A0i):
    return pl.kernel(
        _sc_pass0_body,
        out_type=[
            jax.ShapeDtypeStruct((NQ * N_BONDS, QW), _f32),
            jax.ShapeDtypeStruct((NQ * N_ATOMS, QW), _f32),
            jax.ShapeDtypeStruct((NC * N_BONDS,), _i32),
        ],
        mesh=_mesh,
        compiler_params=pltpu.CompilerParams(use_tc_tiling_on_sc=False),
        scratch_types=[
            pltpu.VMEM((NB, CHUNK), _i32),
            pltpu.VMEM((NB, CHUNK), _i32),
        ] + [pltpu.VMEM((CHUNK,), _i32) for _ in range(NB)]
        + [
            pltpu.VMEM((NB, CHUNK, 16), _i32),
            pltpu.VMEM((NB, CHUNK, QW), _f32),
            pltpu.VMEM((NB, CHUNK, QW), _f32),
            pltpu.VMEM((NB, CHUNK, QW), _f32),
            pltpu.VMEM((DROWS, QW), _f32),
            pltpu.VMEM_SHARED((N_ATOMS, QW), _f32),
        ] + [pltpu.SemaphoreType.DMA for _ in range(5 * NB)],
    )(b2aq, b2a_wide, b2revb, B0q, A0i)


def _sc_depth(b2aq, b2rq, tgt, h0q, AM2q, Hhq, write_h):
    out_type = []
    if write_h:
        out_type.append(jax.ShapeDtypeStruct((NQ * N_BONDS, QW), _f32))
    out_type.append(jax.ShapeDtypeStruct((NQ * N_ATOMS, QW), _f32))
    outs = pl.kernel(
        functools.partial(_sc_depth_body, write_h),
        out_type=out_type,
        mesh=_mesh,
        compiler_params=pltpu.CompilerParams(use_tc_tiling_on_sc=False),
        scratch_types=[
            pltpu.VMEM((NB, CHUNK), _i32),
            pltpu.VMEM((NB, CHUNK), _i32),
        ] + [pltpu.VMEM((CHUNK,), _i32) for _ in range(NB)]
        + [
            pltpu.VMEM((NB, CHUNK, QW), _f32),
            pltpu.VMEM((NB, CHUNK, QW), _f32),
            pltpu.VMEM((NB, CHUNK, QW), _f32),
            pltpu.VMEM((NB, CHUNK, QW), _f32),
            pltpu.VMEM((DROWS, QW), _f32),
            pltpu.VMEM_SHARED((N_ATOMS, QW), _f32),
        ] + [pltpu.SemaphoreType.DMA for _ in range(4 * NB)],
    )(b2aq, b2rq, tgt, h0q, AM2q, Hhq)
    if write_h:
        return outs[0], outs[1]
    return None, (outs[0] if isinstance(outs, (tuple, list)) else outs)


def _qspec2(j, nb, blk):
    return pl.BlockSpec((blk, QW), lambda q, i: (j * nb + i, 0))


def _qspec1(j, nb, blk):
    return pl.BlockSpec((blk, QW), lambda i: (j * nb + i, 0))


def _split_w(wt):
    k = wt.shape[0]
    return wt.reshape(k, NQ, QW).transpose(1, 0, 2)


def _mm_split_body(x_ref, w_ref, o_ref):
    o_ref[...] = jnp.dot(x_ref[...], w_ref[0], preferred_element_type=_f32)


def _mm_split(x, wt, blk):
    n, k = x.shape
    nb = n // blk
    return pl.pallas_call(
        _mm_split_body,
        grid=(NQ, nb),
        in_specs=[
            pl.BlockSpec((blk, k), lambda q, i: (i, 0)),
            pl.BlockSpec((1, k, QW), lambda q, i: (q, 0, 0)),
        ],
        out_specs=pl.BlockSpec((blk, QW), lambda q, i: (q * nb + i, 0)),
        out_shape=jax.ShapeDtypeStruct((NQ * n, QW), _f32),
    )(x, _split_w(wt))


def _mm_full_body(x0_ref, x1_ref, x2_ref, x3_ref, w_ref, o_ref):
    x = jnp.concatenate(
        [x0_ref[...], x1_ref[...], x2_ref[...], x3_ref[...]], axis=1)
    o_ref[...] = jnp.dot(x, w_ref[...], preferred_element_type=_f32)


def _mm_full(xq, wt, n, blk):
    nb = n // blk
    return pl.pallas_call(
        _mm_full_body,
        grid=(nb,),
        in_specs=[_qspec1(j, nb, blk) for j in range(NQ)]
        + [pl.BlockSpec((HIDDEN, HIDDEN), lambda i: (0, 0))],
        out_specs=pl.BlockSpec((blk, HIDDEN), lambda i: (i, 0)),
        out_shape=jax.ShapeDtypeStruct((n, HIDDEN), _f32),
    )(xq, xq, xq, xq, wt)


def _mm_body(x_ref, w_ref, o_ref):
    o_ref[...] = jnp.dot(x_ref[...], w_ref[...], preferred_element_type=_f32)


def _mm(x, wt, blk):
    n, k = x.shape
    h = wt.shape[1]
    return pl.pallas_call(
        _mm_body,
        grid=(n // blk,),
        in_specs=[
            pl.BlockSpec((blk, k), lambda i: (i, 0)),
            pl.BlockSpec((k, h), lambda i: (0, 0)),
        ],
        out_specs=pl.BlockSpec((blk, h), lambda i: (i, 0)),
        out_shape=jax.ShapeDtypeStruct((n, h), _f32),
    )(x, wt)


def _final_body(x0_ref, x1_ref, x2_ref, x3_ref, fa_ref, wm_ref, wa_ref,
                b_ref, ids_ref, o_ref):
    i = pl.program_id(0)
    x = jnp.concatenate(
        [x0_ref[...], x1_ref[...], x2_ref[...], x3_ref[...]], axis=1)
    y = (jnp.dot(x, wm_ref[...], preferred_element_type=_f32)
         + jnp.dot(fa_ref[...], wa_ref[...], preferred_element_type=_f32)
         + b_ref[...])
    atom_h = jnp.maximum(y, 0.0)
    ids = ids_ref[0, 0, :]
    onehot = (lax.broadcasted_iota(_i32, (NMOLP, atom_h.shape[0]), 0)
              == ids[None, :]).astype(_f32)
    contrib = jnp.dot(onehot, atom_h, preferred_element_type=_f32)

    @pl.when(i == 0)
    def _():
        o_ref[...] = jnp.zeros_like(o_ref)

    o_ref[...] += contrib


def _final_readout(Sq, f_atoms, wm, wa, bias, mol_ids, blk=5000):
    nb = N_ATOMS // blk
    ids3 = mol_ids.reshape(nb, 1, blk)
    return pl.pallas_call(
        _final_body,
        grid=(nb,),
        in_specs=[_qspec1(j, nb, blk) for j in range(NQ)] + [
            pl.BlockSpec((blk, ATOM_FDIM), lambda i: (i, 0)),
            pl.BlockSpec((HIDDEN, HIDDEN), lambda i: (0, 0)),
            pl.BlockSpec((ATOM_FDIM, HIDDEN), lambda i: (0, 0)),
            pl.BlockSpec((1, HIDDEN), lambda i: (0, 0)),
            pl.BlockSpec((1, 1, blk), lambda i: (i, 0, 0)),
        ],
        out_specs=pl.BlockSpec((NMOLP, HIDDEN), lambda i: (0, 0)),
        out_shape=jax.ShapeDtypeStruct((NMOLP, HIDDEN), _f32),
    )(Sq, Sq, Sq, Sq, f_atoms, wm, wa, bias.reshape(1, -1), ids3)


def _head_body(mv_ref, r1w_ref, r1b_ref, r2w_ref, r2b_ref, o_ref):
    out = jnp.maximum(
        jnp.dot(mv_ref[...], r1w_ref[...], preferred_element_type=_f32)
        + r1b_ref[...], 0.0)
    o_ref[...] = (
        jnp.dot(out, r2w_ref[...], preferred_element_type=_f32)
        + r2b_ref[...])


def _head(mol_vecs, R1_t, R1_b, R2_t, R2_b):
    m = mol_vecs.shape[0]
    return pl.pallas_call(
        _head_body,
        out_shape=jax.ShapeDtypeStruct((m, TASKS), _f32),
    )(mol_vecs, R1_t, R1_b.reshape(1, -1), R2_t, R2_b.reshape(1, -1))


def kernel(f_atoms, f_bonds, b2a, b2revb, mol_ids, W_i, W_h, W_o_w, W_o_b,
           R1_w, R1_b, R2_w, R2_b):
    W_ia_t = W_i[:, :ATOM_FDIM].T
    W_ib_t = W_i[:, ATOM_FDIM:].T
    W_h_t = W_h.T
    W_oa_t = W_o_w[:, :ATOM_FDIM].T
    W_om_t = W_o_w[:, ATOM_FDIM:].T

    A0i = _mm(f_atoms, W_ia_t, blk=10000).reshape(NQ * N_ATOMS, QW)
    B0q = _mm_split(f_bonds, W_ib_t, blk=20000)
    b2a_wide = jnp.broadcast_to(b2a[:, None], (N_BONDS, 16))
    qoff = jnp.arange(NQ, dtype=_i32)
    b2aq = (NQ * b2a[None, :] + qoff[:, None]).reshape(-1)
    b2rq = (NQ * b2revb[None, :] + qoff[:, None]).reshape(-1)

    h0q, Sq, tgt = _sc_pass0(b2aq, b2a_wide, b2revb, B0q, A0i)

    hq = h0q
    for d in range(DEPTH):
        AM2i = _mm_full(Sq, W_h_t, N_ATOMS, blk=5000).reshape(
            NQ * N_ATOMS, QW)
        Hhi = _mm_full(hq, W_h_t, N_BONDS, blk=5000).reshape(
            NQ * N_BONDS, QW)
        write_h = d < DEPTH - 1
        if write_h:
            hq, Sq = _sc_depth(b2aq, b2rq, tgt, h0q, AM2i, Hhi, True)
        else:
            _, Sq = _sc_depth(b2aq, b2rq, tgt, h0q, AM2i, Hhi, False)

    mol512 = _final_readout(Sq, f_atoms, W_om_t, W_oa_t, W_o_b, mol_ids)
    return _head(mol512[:N_MOLS], R1_w.T, R1_b, R2_w.T, R2_b)
